# Initial kernel scaffold; baseline (speedup 1.0000x reference)
import functools

import jax
import jax.numpy as jnp
from jax import lax
from jax.experimental import pallas as pl
from jax.experimental.pallas import tpu as pltpu
from jax.experimental.pallas import tpu_sc as plsc

N = 100000
E = 1600000
F_IN = 128
HID = 16
K = 3
T = 4

NC, NS, L = 2, 16, 16
NW = NC * NS
N_PAD = 100352
NPT = N_PAD // NS
E_PAD = 1605632
E2 = E_PAD // 128
EPT = E_PAD // NW
MROWS = 28
MEDG = MROWS * 128
NMAC = EPT // MEDG
F32 = jnp.float32
I32 = jnp.int32

_sc_mesh = plsc.VectorSubcoreMesh(core_axis_name="c", subcore_axis_name="s")


def _wid():
    return lax.axis_index("s") * NC + lax.axis_index("c")


def _rsqrt16(v):
    y = plsc.bitcast(jnp.int32(0x5F3759DF) - (plsc.bitcast(v, I32) >> 1), F32)
    for _ in range(3):
        y = y * (1.5 - 0.5 * v * y * y)
    return jnp.where(v > 0.0, y, 0.0)


def _bcast_lane(nv, e):
    return lax.gather(
        nv, jnp.full((L, 1), e, I32),
        lax.GatherDimensionNumbers(
            offset_dims=(), collapsed_slice_dims=(0,), start_index_map=(0,)),
        (1,), mode=lax.GatherScatterMode.PROMISE_IN_BOUNDS)


@functools.partial(
    pl.kernel,
    out_type=jax.ShapeDtypeStruct((NC, N_PAD), F32),
    mesh=_sc_mesh,
    scratch_types=[
        pltpu.VMEM_SHARED((N_PAD,), F32),
        pltpu.VMEM((MROWS, 128), I32),
        pltpu.VMEM((MROWS, 128), F32),
        pltpu.VMEM((NPT,), F32),
        pltpu.SemaphoreType.DMA,
    ],
)
def _s1_deg(dst2d, ea2d, deg_out, sp_deg, idxb, valb, zbuf, sem):
    cid = lax.axis_index("c")
    sid = lax.axis_index("s")
    wid = _wid()

    def zb(i, _):
        zbuf[pl.ds(i * L, L)] = jnp.zeros((L,), F32)
        return 0
    lax.fori_loop(0, NPT // L, zb, 0)
    pltpu.sync_copy(zbuf, sp_deg.at[pl.ds(sid * NPT, NPT)])
    plsc.subcore_barrier()

    row0 = wid * (EPT // 128)

    def macro(m, _):
        base = row0 + m * MROWS
        d1 = pltpu.async_copy(dst2d.at[pl.ds(base, MROWS)], idxb, sem)
        d2 = pltpu.async_copy(ea2d.at[pl.ds(base, MROWS)], valb, sem)
        d1.wait()
        d2.wait()
        descs = []
        for j in range(MROWS):
            descs.append(pltpu.async_copy(
                valb.at[j], sp_deg.at[idxb.at[j]], sem, add=True))
        for d in descs:
            d.wait()
        return 0
    lax.fori_loop(0, NMAC, macro, 0)
    plsc.subcore_barrier()
    pltpu.sync_copy(sp_deg.at[pl.ds(sid * NPT, NPT)],
                    deg_out.at[cid, pl.ds(sid * NPT, NPT)])


@functools.partial(
    pl.kernel,
    out_type=jax.ShapeDtypeStruct((E2, 128), F32),
    mesh=_sc_mesh,
    scratch_types=[
        pltpu.VMEM_SHARED((N_PAD,), F32),
        pltpu.VMEM((MROWS, 128), I32),
        pltpu.VMEM((MROWS, 128), I32),
        pltpu.VMEM((MROWS, 128), F32),
        pltpu.VMEM((MROWS, 128), F32),
        pltpu.VMEM((MROWS, 128), F32),
        pltpu.VMEM((NPT,), F32),
        pltpu.VMEM((NPT,), F32),
        pltpu.SemaphoreType.DMA,
    ],
)
def _s2_norm(deg_parts, src2d, dst2d, ea2d, norm_out,
             sp_dis, sidx, didx, eab, gsb, gdb, dbuf, dbuf2, sem):
    sid = lax.axis_index("s")
    wid = _wid()

    d1 = pltpu.async_copy(deg_parts.at[0, pl.ds(sid * NPT, NPT)], dbuf, sem)
    d2 = pltpu.async_copy(deg_parts.at[1, pl.ds(sid * NPT, NPT)], dbuf2, sem)
    d1.wait()
    d2.wait()

    def mk(i, _):
        s = pl.ds(i * L, L)
        dbuf[s] = _rsqrt16(dbuf[s] + dbuf2[s])
        return 0
    lax.fori_loop(0, NPT // L, mk, 0)
    pltpu.sync_copy(dbuf, sp_dis.at[pl.ds(sid * NPT, NPT)])
    plsc.subcore_barrier()

    row0 = wid * (EPT // 128)

    def macro(m, _):
        base = row0 + m * MROWS
        c1 = pltpu.async_copy(src2d.at[pl.ds(base, MROWS)], sidx, sem)
        c2 = pltpu.async_copy(dst2d.at[pl.ds(base, MROWS)], didx, sem)
        c3 = pltpu.async_copy(ea2d.at[pl.ds(base, MROWS)], eab, sem)
        c1.wait()
        c2.wait()
        c3.wait()
        descs = []
        for j in range(MROWS):
            descs.append(pltpu.async_copy(sp_dis.at[sidx.at[j]], gsb.at[j], sem))
            descs.append(pltpu.async_copy(sp_dis.at[didx.at[j]], gdb.at[j], sem))
        for d in descs:
            d.wait()

        def mul(r, _):
            for c in range(8):
                s = pl.ds(c * L, L)
                eab[r, s] = gsb[r, s] * eab[r, s] * gdb[r, s]
            return 0
        lax.fori_loop(0, MROWS, mul, 0)
        pltpu.sync_copy(eab, norm_out.at[pl.ds(base, MROWS)])
        return 0
    lax.fori_loop(0, NMAC, macro, 0)


@functools.partial(
    pl.kernel,
    out_type=jax.ShapeDtypeStruct((NC, K, N_PAD, HID), F32),
    mesh=_sc_mesh,
    scratch_types=[
        pltpu.VMEM_SHARED((N_PAD, HID), F32),
        pltpu.VMEM((MROWS, 128), I32),
        pltpu.VMEM((MROWS, 128), I32),
        pltpu.VMEM((MROWS, 128), I32),
        pltpu.VMEM((MROWS, 128), F32),
        pltpu.VMEM((MEDG, HID), F32),
        pltpu.VMEM((NPT // 8, HID), F32),
        pltpu.SemaphoreType.DMA,
        pltpu.SemaphoreType.DMA,
    ],
)
def _s3_msg(h_tab, src2d, dst2d, norm2d, parts,
            sp_agg, sidx, gidx, didx, nrm, rows, zbuf, sem, sem2):
    cid = lax.axis_index("c")
    sid = lax.axis_index("s")
    wid = _wid()
    row0 = wid * (EPT // 128)

    def zb(i, _):
        zbuf[i, :] = jnp.zeros((L,), F32)
        return 0
    lax.fori_loop(0, NPT // 8, zb, 0)

    for k in range(K):
        koff = jnp.int32(k * N_PAD)
        for z in range(8):
            pltpu.sync_copy(
                zbuf, sp_agg.at[pl.ds(sid * NPT + z * (NPT // 8), NPT // 8)])
        plsc.subcore_barrier()

        def macro(m, _):
            base = row0 + m * MROWS
            c1 = pltpu.async_copy(src2d.at[pl.ds(base, MROWS)], sidx, sem)
            c2 = pltpu.async_copy(dst2d.at[pl.ds(base, MROWS)], didx, sem)
            c3 = pltpu.async_copy(norm2d.at[pl.ds(base, MROWS)], nrm, sem)
            c1.wait()

            def addk(r, _):
                for c in range(8):
                    s = pl.ds(c * L, L)
                    gidx[r, s] = sidx[r, s] + koff
                return 0
            lax.fori_loop(0, MROWS, addk, 0)
            gd = []
            for j in range(MROWS):
                gd.append(pltpu.async_copy(
                    h_tab.at[gidx.at[j]], rows.at[pl.ds(j * 128, 128)], sem2))
            c2.wait()
            c3.wait()
            for d in gd:
                d.wait()

            def scale(r, _):
                for c in range(8):
                    nv = nrm[r, pl.ds(c * L, L)]
                    for e in range(L):
                        i = r * 128 + c * L + e
                        rows[i, :] = rows[i, :] * _bcast_lane(nv, e)
                return 0
            lax.fori_loop(0, MROWS, scale, 0)

            sc = []
            for j in range(MROWS):
                sc.append(pltpu.async_copy(
                    rows.at[pl.ds(j * 128, 128)], sp_agg.at[didx.at[j]],
                    sem2, add=True))
            for d in sc:
                d.wait()
            return 0
        lax.fori_loop(0, NMAC, macro, 0)
        plsc.subcore_barrier()
        pltpu.sync_copy(sp_agg.at[pl.ds(sid * NPT, NPT)],
                        parts.at[cid, k, pl.ds(sid * NPT, NPT)])
        plsc.subcore_barrier()


@functools.partial(
    pl.kernel,
    out_type=jax.ShapeDtypeStruct((NC, K * N_PAD), F32),
    mesh=_sc_mesh,
    scratch_types=[
        pltpu.VMEM_SHARED((K * N_PAD,), F32),
        pltpu.VMEM_SHARED((K * N_PAD,), F32),
        pltpu.VMEM((MROWS, 128), I32),
        pltpu.VMEM((MROWS, 128), I32),
        pltpu.VMEM((MROWS, 128), I32),
        pltpu.VMEM((MROWS, 128), F32),
        pltpu.VMEM((MROWS, 128), F32),
        pltpu.VMEM((NPT,), F32),
        pltpu.VMEM((NPT,), F32),
        pltpu.VMEM((NPT,), F32),
        pltpu.VMEM((L,), F32),
        pltpu.SemaphoreType.DMA,
        pltpu.SemaphoreType.DMA,
    ],
)
def _s4_msg(pprev, cvec, w2row, src2d, dst2d, norm2d, parts,
            sp_h2, sp_agg, sidx, gidx, didx, nrm, vals,
            p0b, p1b, cb, wb, sem, sem2):
    cid = lax.axis_index("c")
    sid = lax.axis_index("s")
    wid = _wid()
    row0 = wid * (EPT // 128)

    def zb(i, _):
        p0b[pl.ds(i * L, L)] = jnp.zeros((L,), F32)
        return 0
    for k in range(K):
        off = k * N_PAD + sid * NPT
        pltpu.sync_copy(w2row.at[pl.ds(k * L, L)], wb)
        d1 = pltpu.async_copy(pprev.at[0, pl.ds(off, NPT)], p0b, sem)
        d2 = pltpu.async_copy(pprev.at[1, pl.ds(off, NPT)], p1b, sem)
        d3 = pltpu.async_copy(cvec.at[pl.ds(off, NPT)], cb, sem)
        d1.wait()
        d2.wait()
        d3.wait()
        wv = wb[pl.ds(0, L)]

        def mk(i, _):
            s = pl.ds(i * L, L)
            cb[s] = wv * (p0b[s] + p1b[s]) + cb[s]
            return 0
        lax.fori_loop(0, NPT // L, mk, 0)
        pltpu.sync_copy(cb, sp_h2.at[pl.ds(off, NPT)])
        lax.fori_loop(0, NPT // L, zb, 0)
        pltpu.sync_copy(p0b, sp_agg.at[pl.ds(off, NPT)])
    plsc.subcore_barrier()

    def macro(m, _):
        base = row0 + m * MROWS
        c1 = pltpu.async_copy(src2d.at[pl.ds(base, MROWS)], sidx, sem)
        c2 = pltpu.async_copy(dst2d.at[pl.ds(base, MROWS)], didx, sem)
        c3 = pltpu.async_copy(norm2d.at[pl.ds(base, MROWS)], nrm, sem)
        c1.wait()
        c2.wait()
        c3.wait()
        for k in range(K):
            koff = jnp.int32(k * N_PAD)

            def addk(r, _):
                for c in range(8):
                    s = pl.ds(c * L, L)
                    gidx[r, s] = sidx[r, s] + koff
                return 0
            lax.fori_loop(0, MROWS, addk, 0)
            gd = []
            for j in range(MROWS):
                gd.append(pltpu.async_copy(
                    sp_h2.at[gidx.at[j]], vals.at[j], sem2))
            for d in gd:
                d.wait()

            def mul(r, _):
                for c in range(8):
                    s = pl.ds(c * L, L)
                    vals[r, s] = vals[r, s] * nrm[r, s]
                    gidx[r, s] = didx[r, s] + koff
                return 0
            lax.fori_loop(0, MROWS, mul, 0)
            sc = []
            for j in range(MROWS):
                sc.append(pltpu.async_copy(
                    vals.at[j], sp_agg.at[gidx.at[j]], sem2, add=True))
            for d in sc:
                d.wait()
        return 0
    lax.fori_loop(0, NMAC, macro, 0)
    plsc.subcore_barrier()
    for k in range(K):
        off = k * N_PAD + sid * NPT
        pltpu.sync_copy(sp_agg.at[pl.ds(off, NPT)],
                        parts.at[cid, pl.ds(off, NPT)])


_BN_ROWS = 512
_NBLK = N_PAD // _BN_ROWS


def _t1_body(x_ref, rw_ref, iw_ref, b_ref, root_ref, h0_ref):
    xb = x_ref[...]
    for k in range(K):
        root_ref[k] = jnp.dot(xb, rw_ref[k], preferred_element_type=F32) \
            + b_ref[k]
        h0_ref[k] = jnp.dot(xb, iw_ref[k], preferred_element_type=F32)


def _t1_call(x, rw, iw, b):
    return pl.pallas_call(
        _t1_body,
        grid=(_NBLK,),
        in_specs=[
            pl.BlockSpec((_BN_ROWS, F_IN), lambda i: (i, 0)),
            pl.BlockSpec((K, F_IN, HID), lambda i: (0, 0, 0)),
            pl.BlockSpec((K, F_IN, HID), lambda i: (0, 0, 0)),
            pl.BlockSpec((K, 1, HID), lambda i: (0, 0, 0)),
        ],
        out_specs=[
            pl.BlockSpec((K, _BN_ROWS, HID), lambda i: (0, i, 0)),
            pl.BlockSpec((K, _BN_ROWS, HID), lambda i: (0, i, 0)),
        ],
        out_shape=[
            jax.ShapeDtypeStruct((K, N_PAD, HID), F32),
            jax.ShapeDtypeStruct((K, N_PAD, HID), F32),
        ],
    )(x, rw, iw, b)


def _t2_body(p_ref, root_ref, w_ref, h_ref):
    for k in range(K):
        out = jnp.maximum(p_ref[0, k] + p_ref[1, k] + root_ref[k], 0.0)
        h_ref[k] = jnp.dot(out, w_ref[k], preferred_element_type=F32)


def _t2_call(parts, rootb, w):
    return pl.pallas_call(
        _t2_body,
        grid=(_NBLK,),
        in_specs=[
            pl.BlockSpec((NC, K, _BN_ROWS, HID), lambda i: (0, 0, i, 0)),
            pl.BlockSpec((K, _BN_ROWS, HID), lambda i: (0, i, 0)),
            pl.BlockSpec((K, HID, HID), lambda i: (0, 0, 0)),
        ],
        out_specs=pl.BlockSpec((K, _BN_ROWS, HID), lambda i: (0, i, 0)),
        out_shape=jax.ShapeDtypeStruct((K, N_PAD, HID), F32),
    )(parts, rootb, w)


def _t2b_body(p_ref, root_ref, g_ref, be_ref, mu_ref, var_ref,
              rw2_ref, iw2_ref, b2_ref, w2_ref, root2_ref, h20_ref, c1_ref):
    acc = jnp.zeros((_BN_ROWS, HID), F32)
    for k in range(K):
        acc = acc + jnp.maximum(p_ref[0, k] + p_ref[1, k] + root_ref[k], 0.0)
    hm = acc * (1.0 / K)
    scale = g_ref[0] * lax.rsqrt(var_ref[0] + 1e-5)
    hbn = jnp.maximum((hm - mu_ref[0]) * scale + be_ref[0], 0.0)
    for k in range(K):
        r2 = jnp.dot(hbn, rw2_ref[k], preferred_element_type=F32)[:, 0] \
            + b2_ref[k, 0, 0]
        root2_ref[k] = r2
        h20_ref[k] = jnp.dot(hbn, iw2_ref[k], preferred_element_type=F32)[:, 0]
        c1_ref[k] = r2 * w2_ref[k, 0, 0]


def _t2b_call(parts, rootb, bn_g, bn_b, bn_mu, bn_var, rw2, iw2, b2, w2):
    return pl.pallas_call(
        _t2b_body,
        grid=(_NBLK,),
        in_specs=[
            pl.BlockSpec((NC, K, _BN_ROWS, HID), lambda i: (0, 0, i, 0)),
            pl.BlockSpec((K, _BN_ROWS, HID), lambda i: (0, i, 0)),
            pl.BlockSpec((1, HID), lambda i: (0, 0)),
            pl.BlockSpec((1, HID), lambda i: (0, 0)),
            pl.BlockSpec((1, HID), lambda i: (0, 0)),
            pl.BlockSpec((1, HID), lambda i: (0, 0)),
            pl.BlockSpec((K, HID, 1), lambda i: (0, 0, 0)),
            pl.BlockSpec((K, HID, 1), lambda i: (0, 0, 0)),
            pl.BlockSpec((K, 1, 1), lambda i: (0, 0, 0)),
            pl.BlockSpec((K, 1, 1), lambda i: (0, 0, 0)),
        ],
        out_specs=[
            pl.BlockSpec((K, _BN_ROWS), lambda i: (0, i)),
            pl.BlockSpec((K, _BN_ROWS), lambda i: (0, i)),
            pl.BlockSpec((K, _BN_ROWS), lambda i: (0, i)),
        ],
        out_shape=[
            jax.ShapeDtypeStruct((K, N_PAD), F32),
            jax.ShapeDtypeStruct((K, N_PAD), F32),
            jax.ShapeDtypeStruct((K, N_PAD), F32),
        ],
    )(parts, rootb, bn_g, bn_b, bn_mu, bn_var, rw2, iw2, b2, w2)


def _f1_body(p_ref, root2_ref, o_ref):
    s = jnp.zeros((_BN_ROWS,), F32)
    for k in range(K):
        s = s + p_ref[0, k] + p_ref[1, k] + root2_ref[k]
    o_ref[...] = jax.nn.sigmoid(s * (1.0 / K))


def _f1_call(parts, root2b):
    return pl.pallas_call(
        _f1_body,
        grid=(_NBLK,),
        in_specs=[
            pl.BlockSpec((NC, K, _BN_ROWS), lambda i: (0, 0, i)),
            pl.BlockSpec((K, _BN_ROWS), lambda i: (0, i)),
        ],
        out_specs=pl.BlockSpec((_BN_ROWS,), lambda i: (i,)),
        out_shape=jax.ShapeDtypeStruct((N_PAD,), F32),
    )(parts, root2b)


def kernel(x, edge_index, edge_attr, batch,
           conv1_init_w, conv1_w, conv1_root_w, conv1_bias,
           bn1_gamma, bn1_beta, bn1_mean, bn1_var,
           conv2_init_w, conv2_w, conv2_root_w, conv2_bias):
    del batch
    pad = E_PAD - E
    fill = (jnp.arange(pad, dtype=I32) * 37) % N
    src = jnp.concatenate([edge_index[0].astype(I32), fill]).reshape(E2, 128)
    dst = jnp.concatenate([edge_index[1].astype(I32), fill]).reshape(E2, 128)
    ea = jnp.concatenate([edge_attr.astype(F32),
                          jnp.zeros((pad,), F32)]).reshape(E2, 128)

    deg_parts = _s1_deg(dst, ea)
    norm2d = _s2_norm(deg_parts, src, dst, ea)

    rootb, h0 = _t1_call(x, conv1_root_w, conv1_init_w, conv1_bias)

    h = h0.reshape(K * N_PAD, HID)
    for _ in range(T - 1):
        parts1 = _s3_msg(h, src, dst, norm2d)
        h = _t2_call(parts1, rootb, conv1_w).reshape(K * N_PAD, HID)
    parts1 = _s3_msg(h, src, dst, norm2d)
    root2b, h20, c1 = _t2b_call(
        parts1, rootb, bn1_gamma.reshape(1, HID), bn1_beta.reshape(1, HID),
        bn1_mean.reshape(1, HID), bn1_var.reshape(1, HID),
        conv2_root_w, conv2_init_w, conv2_bias, conv2_w)

    w2row = jnp.broadcast_to(
        conv2_w.reshape(K, 1).astype(F32), (K, L)).reshape(K * L)
    zeros2 = jnp.zeros((NC, K * N_PAD), F32)
    cvec = h20.reshape(K * N_PAD)
    c1f = c1.reshape(K * N_PAD)
    parts2 = _s4_msg(zeros2, cvec, w2row, src, dst, norm2d)
    for _ in range(T - 1):
        parts2 = _s4_msg(parts2, c1f, w2row, src, dst, norm2d)

    out = _f1_call(parts2, root2b.reshape(K, N_PAD))
    return out[:N].reshape(N, 1)


# speedup vs baseline: 132.4642x; 132.4642x over previous
import functools

import jax
import jax.numpy as jnp
from jax import lax
from jax.experimental import pallas as pl
from jax.experimental.pallas import tpu as pltpu
from jax.experimental.pallas import tpu_sc as plsc

N = 100000
E = 1600000
F_IN = 128
HID = 16
K = 3
T = 4

NC, NS, L = 2, 16, 16
NW = NC * NS
N_PAD = 100352
NPT = N_PAD // NS
E_PAD = 1605632
E2 = E_PAD // 128
EPT = E_PAD // NW
MROWS = 56
MEDG = MROWS * 128
NMAC = EPT // MEDG
MR3 = 8
MEDG3 = MR3 * 128
NMAC3 = EPT // MEDG3
F32 = jnp.float32
I32 = jnp.int32

_sc_mesh = plsc.VectorSubcoreMesh(core_axis_name="c", subcore_axis_name="s")


def _wid():
    return lax.axis_index("s") * NC + lax.axis_index("c")


def _bcast_lane(nv, e):
    return lax.gather(
        nv, jnp.full((L, 1), e, I32),
        lax.GatherDimensionNumbers(
            offset_dims=(), collapsed_slice_dims=(0,), start_index_map=(0,)),
        (1,), mode=lax.GatherScatterMode.PROMISE_IN_BOUNDS)


@functools.partial(
    pl.kernel,
    out_type=jax.ShapeDtypeStruct((NC, N_PAD), F32),
    mesh=_sc_mesh,
    compiler_params=pltpu.CompilerParams(use_tc_tiling_on_sc=False),
    scratch_types=[
        pltpu.VMEM_SHARED((N_PAD,), F32),
        pltpu.VMEM((MROWS, 128), I32),
        pltpu.VMEM((MROWS, 128), F32),
        pltpu.VMEM((NPT,), F32),
        pltpu.SemaphoreType.DMA,
    ],
)
def _s1_deg(dst2d, ea2d, deg_out, sp_deg, idxb, valb, zbuf, sem):
    cid = lax.axis_index("c")
    sid = lax.axis_index("s")
    wid = _wid()

    def zb(i, _):
        zbuf[pl.ds(i * L, L)] = jnp.zeros((L,), F32)
        return 0
    lax.fori_loop(0, NPT // L, zb, 0)
    pltpu.sync_copy(zbuf, sp_deg.at[pl.ds(sid * NPT, NPT)])
    plsc.subcore_barrier()

    row0 = wid * (EPT // 128)

    def macro(m, _):
        base = row0 + m * MROWS
        d1 = pltpu.async_copy(dst2d.at[pl.ds(base, MROWS)], idxb, sem)
        d2 = pltpu.async_copy(ea2d.at[pl.ds(base, MROWS)], valb, sem)
        d1.wait()
        d2.wait()
        descs = []
        for j in range(MROWS):
            descs.append(pltpu.async_copy(
                valb.at[j], sp_deg.at[idxb.at[j]], sem, add=True))
        for d in descs:
            d.wait()
        return 0
    lax.fori_loop(0, NMAC, macro, 0)
    plsc.subcore_barrier()
    pltpu.sync_copy(sp_deg.at[pl.ds(sid * NPT, NPT)],
                    deg_out.at[cid, pl.ds(sid * NPT, NPT)])


@functools.partial(
    pl.kernel,
    out_type=jax.ShapeDtypeStruct((E2, 128), F32),
    mesh=_sc_mesh,
    compiler_params=pltpu.CompilerParams(use_tc_tiling_on_sc=False),
    scratch_types=[
        pltpu.VMEM_SHARED((N_PAD,), F32),
        pltpu.VMEM((MROWS, 128), I32),
        pltpu.VMEM((MROWS, 128), I32),
        pltpu.VMEM((MROWS, 128), F32),
        pltpu.VMEM((MROWS, 128), F32),
        pltpu.VMEM((MROWS, 128), F32),
        pltpu.SemaphoreType.DMA,
    ],
)
def _s2_norm(dis_tab, src2d, dst2d, ea2d, norm_out,
             sp_dis, sidx, didx, eab, gsb, gdb, sem):
    sid = lax.axis_index("s")
    wid = _wid()

    pltpu.sync_copy(dis_tab.at[pl.ds(sid * NPT, NPT)],
                    sp_dis.at[pl.ds(sid * NPT, NPT)])
    plsc.subcore_barrier()

    row0 = wid * (EPT // 128)

    def macro(m, _):
        base = row0 + m * MROWS
        c1 = pltpu.async_copy(src2d.at[pl.ds(base, MROWS)], sidx, sem)
        c2 = pltpu.async_copy(dst2d.at[pl.ds(base, MROWS)], didx, sem)
        c3 = pltpu.async_copy(ea2d.at[pl.ds(base, MROWS)], eab, sem)
        c1.wait()
        c2.wait()
        c3.wait()
        descs = []
        for j in range(MROWS):
            descs.append(pltpu.async_copy(sp_dis.at[sidx.at[j]], gsb.at[j], sem))
            descs.append(pltpu.async_copy(sp_dis.at[didx.at[j]], gdb.at[j], sem))
        for d in descs:
            d.wait()

        def mul(r, _):
            for c in range(8):
                s = pl.ds(c * L, L)
                eab[r, s] = gsb[r, s] * eab[r, s] * gdb[r, s]
            return 0
        lax.fori_loop(0, MROWS, mul, 0)
        pltpu.sync_copy(eab, norm_out.at[pl.ds(base, MROWS)])
        return 0
    lax.fori_loop(0, NMAC, macro, 0)


@functools.partial(
    pl.kernel,
    out_type=jax.ShapeDtypeStruct((NC, K, N_PAD, HID), F32),
    mesh=_sc_mesh,
    compiler_params=pltpu.CompilerParams(use_tc_tiling_on_sc=False),
    scratch_types=[
        pltpu.VMEM_SHARED((N_PAD, HID), F32),
        pltpu.VMEM((MR3, 128), I32),
        pltpu.VMEM((MR3, 128), I32),
        pltpu.VMEM((MR3, 128), I32),
        pltpu.VMEM((MR3, 128), F32),
        pltpu.VMEM((MEDG3, HID), F32),
        pltpu.VMEM((NPT // 16, HID), F32),
        pltpu.SemaphoreType.DMA,
        pltpu.SemaphoreType.DMA,
    ],
)
def _s3_msg(h_tab, src2d, dst2d, norm2d, parts,
            sp_agg, sidx, gidx, didx, nrm, rows, zbuf, sem, sem2):
    cid = lax.axis_index("c")
    sid = lax.axis_index("s")
    wid = _wid()
    row0 = wid * (EPT // 128)

    def zb(i, _):
        zbuf[i, :] = jnp.zeros((L,), F32)
        return 0
    lax.fori_loop(0, NPT // 16, zb, 0)

    for k in range(K):
        koff = jnp.int32(k * N_PAD)
        for z in range(16):
            pltpu.sync_copy(
                zbuf, sp_agg.at[pl.ds(sid * NPT + z * (NPT // 16), NPT // 16)])
        plsc.subcore_barrier()

        def macro(m, _):
            base = row0 + m * MR3
            c1 = pltpu.async_copy(src2d.at[pl.ds(base, MR3)], sidx, sem)
            c2 = pltpu.async_copy(dst2d.at[pl.ds(base, MR3)], didx, sem)
            c3 = pltpu.async_copy(norm2d.at[pl.ds(base, MR3)], nrm, sem)
            c1.wait()

            def addk(r, _):
                for c in range(8):
                    s = pl.ds(c * L, L)
                    gidx[r, s] = sidx[r, s] + koff
                return 0
            lax.fori_loop(0, MR3, addk, 0)
            gd = []
            for j in range(MR3):
                gd.append(pltpu.async_copy(
                    h_tab.at[gidx.at[j]], rows.at[pl.ds(j * 128, 128)], sem2))
            c2.wait()
            c3.wait()
            for d in gd:
                d.wait()

            def scale(r, _):
                for c in range(8):
                    nv = nrm[r, pl.ds(c * L, L)]
                    for e in range(L):
                        i = r * 128 + c * L + e
                        rows[i, :] = rows[i, :] * _bcast_lane(nv, e)
                return 0
            lax.fori_loop(0, MR3, scale, 0)

            sc = []
            for j in range(MR3):
                sc.append(pltpu.async_copy(
                    rows.at[pl.ds(j * 128, 128)], sp_agg.at[didx.at[j]],
                    sem2, add=True))
            for d in sc:
                d.wait()
            return 0
        lax.fori_loop(0, NMAC3, macro, 0)
        plsc.subcore_barrier()
        pltpu.sync_copy(sp_agg.at[pl.ds(sid * NPT, NPT)],
                        parts.at[cid, k, pl.ds(sid * NPT, NPT)])
        plsc.subcore_barrier()


@functools.partial(
    pl.kernel,
    out_type=jax.ShapeDtypeStruct((NC, K * N_PAD), F32),
    mesh=_sc_mesh,
    compiler_params=pltpu.CompilerParams(use_tc_tiling_on_sc=False),
    scratch_types=[
        pltpu.VMEM_SHARED((K * N_PAD,), F32),
        pltpu.VMEM_SHARED((K * N_PAD,), F32),
        pltpu.VMEM((MROWS, 128), I32),
        pltpu.VMEM((MROWS, 128), I32),
        pltpu.VMEM((MROWS, 128), I32),
        pltpu.VMEM((MROWS, 128), F32),
        pltpu.VMEM((MROWS, 128), F32),
        pltpu.VMEM((NPT,), F32),
        pltpu.VMEM((NPT,), F32),
        pltpu.VMEM((NPT,), F32),
        pltpu.VMEM((L,), F32),
        pltpu.SemaphoreType.DMA,
        pltpu.SemaphoreType.DMA,
    ],
)
def _s4_msg(pprev, cvec, w2row, src2d, dst2d, norm2d, parts,
            sp_h2, sp_agg, sidx, gidx, didx, nrm, vals,
            p0b, p1b, cb, wb, sem, sem2):
    cid = lax.axis_index("c")
    sid = lax.axis_index("s")
    wid = _wid()
    row0 = wid * (EPT // 128)

    def zb(i, _):
        p0b[pl.ds(i * L, L)] = jnp.zeros((L,), F32)
        return 0
    for k in range(K):
        off = k * N_PAD + sid * NPT
        pltpu.sync_copy(w2row.at[pl.ds(k * L, L)], wb)
        d1 = pltpu.async_copy(pprev.at[0, pl.ds(off, NPT)], p0b, sem)
        d2 = pltpu.async_copy(pprev.at[1, pl.ds(off, NPT)], p1b, sem)
        d3 = pltpu.async_copy(cvec.at[pl.ds(off, NPT)], cb, sem)
        d1.wait()
        d2.wait()
        d3.wait()
        wv = wb[pl.ds(0, L)]

        def mk(i, _):
            s = pl.ds(i * L, L)
            cb[s] = wv * (p0b[s] + p1b[s]) + cb[s]
            return 0
        lax.fori_loop(0, NPT // L, mk, 0)
        pltpu.sync_copy(cb, sp_h2.at[pl.ds(off, NPT)])
        lax.fori_loop(0, NPT // L, zb, 0)
        pltpu.sync_copy(p0b, sp_agg.at[pl.ds(off, NPT)])
    plsc.subcore_barrier()

    def macro(m, _):
        base = row0 + m * MROWS
        c1 = pltpu.async_copy(src2d.at[pl.ds(base, MROWS)], sidx, sem)
        c2 = pltpu.async_copy(dst2d.at[pl.ds(base, MROWS)], didx, sem)
        c3 = pltpu.async_copy(norm2d.at[pl.ds(base, MROWS)], nrm, sem)
        c1.wait()
        c2.wait()
        c3.wait()
        for k in range(K):
            koff = jnp.int32(k * N_PAD)

            def addk(r, _):
                for c in range(8):
                    s = pl.ds(c * L, L)
                    gidx[r, s] = sidx[r, s] + koff
                return 0
            lax.fori_loop(0, MROWS, addk, 0)
            gd = []
            for j in range(MROWS):
                gd.append(pltpu.async_copy(
                    sp_h2.at[gidx.at[j]], vals.at[j], sem2))
            for d in gd:
                d.wait()

            def mul(r, _):
                for c in range(8):
                    s = pl.ds(c * L, L)
                    vals[r, s] = vals[r, s] * nrm[r, s]
                    gidx[r, s] = didx[r, s] + koff
                return 0
            lax.fori_loop(0, MROWS, mul, 0)
            sc = []
            for j in range(MROWS):
                sc.append(pltpu.async_copy(
                    vals.at[j], sp_agg.at[gidx.at[j]], sem2, add=True))
            for d in sc:
                d.wait()
        return 0
    lax.fori_loop(0, NMAC, macro, 0)
    plsc.subcore_barrier()
    for k in range(K):
        off = k * N_PAD + sid * NPT
        pltpu.sync_copy(sp_agg.at[pl.ds(off, NPT)],
                        parts.at[cid, pl.ds(off, NPT)])


_BN_ROWS = 512
_NBLK = N_PAD // _BN_ROWS


def _t1_body(x_ref, rw_ref, iw_ref, b_ref, deg_ref, root_ref, h0_ref,
             dis_ref):
    xb = x_ref[...]
    for k in range(K):
        root_ref[k] = jnp.dot(xb, rw_ref[k], preferred_element_type=F32) \
            + b_ref[k]
        h0_ref[k] = jnp.dot(xb, iw_ref[k], preferred_element_type=F32)
    d = deg_ref[0] + deg_ref[1]
    dis_ref[...] = jnp.where(d > 0.0, lax.rsqrt(jnp.abs(d) + 1e-30), 0.0)


def _t1_call(x, rw, iw, b, deg_parts):
    return pl.pallas_call(
        _t1_body,
        grid=(_NBLK,),
        in_specs=[
            pl.BlockSpec((_BN_ROWS, F_IN), lambda i: (i, 0)),
            pl.BlockSpec((K, F_IN, HID), lambda i: (0, 0, 0)),
            pl.BlockSpec((K, F_IN, HID), lambda i: (0, 0, 0)),
            pl.BlockSpec((K, 1, HID), lambda i: (0, 0, 0)),
            pl.BlockSpec((NC, _BN_ROWS), lambda i: (0, i)),
        ],
        out_specs=[
            pl.BlockSpec((K, _BN_ROWS, HID), lambda i: (0, i, 0)),
            pl.BlockSpec((K, _BN_ROWS, HID), lambda i: (0, i, 0)),
            pl.BlockSpec((_BN_ROWS,), lambda i: (i,)),
        ],
        out_shape=[
            jax.ShapeDtypeStruct((K, N_PAD, HID), F32),
            jax.ShapeDtypeStruct((K, N_PAD, HID), F32),
            jax.ShapeDtypeStruct((N_PAD,), F32),
        ],
    )(x, rw, iw, b, deg_parts)


def _t2_body(p_ref, root_ref, w_ref, h_ref):
    for k in range(K):
        out = jnp.maximum(p_ref[0, k] + p_ref[1, k] + root_ref[k], 0.0)
        h_ref[k] = jnp.dot(out, w_ref[k], preferred_element_type=F32)


def _t2_call(parts, rootb, w):
    return pl.pallas_call(
        _t2_body,
        grid=(_NBLK,),
        in_specs=[
            pl.BlockSpec((NC, K, _BN_ROWS, HID), lambda i: (0, 0, i, 0)),
            pl.BlockSpec((K, _BN_ROWS, HID), lambda i: (0, i, 0)),
            pl.BlockSpec((K, HID, HID), lambda i: (0, 0, 0)),
        ],
        out_specs=pl.BlockSpec((K, _BN_ROWS, HID), lambda i: (0, i, 0)),
        out_shape=jax.ShapeDtypeStruct((K, N_PAD, HID), F32),
    )(parts, rootb, w)


def _t2b_body(p_ref, root_ref, g_ref, be_ref, mu_ref, var_ref,
              rw2_ref, iw2_ref, b2_ref, w2_ref, root2_ref, h20_ref, c1_ref):
    acc = jnp.zeros((_BN_ROWS, HID), F32)
    for k in range(K):
        acc = acc + jnp.maximum(p_ref[0, k] + p_ref[1, k] + root_ref[k], 0.0)
    hm = acc * (1.0 / K)
    scale = g_ref[0] * lax.rsqrt(var_ref[0] + 1e-5)
    hbn = jnp.maximum((hm - mu_ref[0]) * scale + be_ref[0], 0.0)
    for k in range(K):
        r2 = jnp.dot(hbn, rw2_ref[k], preferred_element_type=F32)[:, 0] \
            + b2_ref[k, 0, 0]
        root2_ref[k] = r2
        h20_ref[k] = jnp.dot(hbn, iw2_ref[k], preferred_element_type=F32)[:, 0]
        c1_ref[k] = r2 * w2_ref[k, 0, 0]


def _t2b_call(parts, rootb, bn_g, bn_b, bn_mu, bn_var, rw2, iw2, b2, w2):
    return pl.pallas_call(
        _t2b_body,
        grid=(_NBLK,),
        in_specs=[
            pl.BlockSpec((NC, K, _BN_ROWS, HID), lambda i: (0, 0, i, 0)),
            pl.BlockSpec((K, _BN_ROWS, HID), lambda i: (0, i, 0)),
            pl.BlockSpec((1, HID), lambda i: (0, 0)),
            pl.BlockSpec((1, HID), lambda i: (0, 0)),
            pl.BlockSpec((1, HID), lambda i: (0, 0)),
            pl.BlockSpec((1, HID), lambda i: (0, 0)),
            pl.BlockSpec((K, HID, 1), lambda i: (0, 0, 0)),
            pl.BlockSpec((K, HID, 1), lambda i: (0, 0, 0)),
            pl.BlockSpec((K, 1, 1), lambda i: (0, 0, 0)),
            pl.BlockSpec((K, 1, 1), lambda i: (0, 0, 0)),
        ],
        out_specs=[
            pl.BlockSpec((K, _BN_ROWS), lambda i: (0, i)),
            pl.BlockSpec((K, _BN_ROWS), lambda i: (0, i)),
            pl.BlockSpec((K, _BN_ROWS), lambda i: (0, i)),
        ],
        out_shape=[
            jax.ShapeDtypeStruct((K, N_PAD), F32),
            jax.ShapeDtypeStruct((K, N_PAD), F32),
            jax.ShapeDtypeStruct((K, N_PAD), F32),
        ],
    )(parts, rootb, bn_g, bn_b, bn_mu, bn_var, rw2, iw2, b2, w2)


def _f1_body(p_ref, root2_ref, o_ref):
    s = jnp.zeros((_BN_ROWS,), F32)
    for k in range(K):
        s = s + p_ref[0, k] + p_ref[1, k] + root2_ref[k]
    o_ref[...] = jax.nn.sigmoid(s * (1.0 / K))


def _f1_call(parts, root2b):
    return pl.pallas_call(
        _f1_body,
        grid=(_NBLK,),
        in_specs=[
            pl.BlockSpec((NC, K, _BN_ROWS), lambda i: (0, 0, i)),
            pl.BlockSpec((K, _BN_ROWS), lambda i: (0, i)),
        ],
        out_specs=pl.BlockSpec((_BN_ROWS,), lambda i: (i,)),
        out_shape=jax.ShapeDtypeStruct((N_PAD,), F32),
    )(parts, root2b)


def kernel(x, edge_index, edge_attr, batch,
           conv1_init_w, conv1_w, conv1_root_w, conv1_bias,
           bn1_gamma, bn1_beta, bn1_mean, bn1_var,
           conv2_init_w, conv2_w, conv2_root_w, conv2_bias):
    del batch
    pad = E_PAD - E
    fill = (jnp.arange(pad, dtype=I32) * 37) % N
    src = jnp.concatenate([edge_index[0].astype(I32), fill]).reshape(E2, 128)
    dst = jnp.concatenate([edge_index[1].astype(I32), fill]).reshape(E2, 128)
    ea = jnp.concatenate([edge_attr.astype(F32),
                          jnp.zeros((pad,), F32)]).reshape(E2, 128)

    deg_parts = _s1_deg(dst, ea)
    rootb, h0, dis = _t1_call(x, conv1_root_w, conv1_init_w, conv1_bias,
                              deg_parts)
    norm2d = _s2_norm(dis, src, dst, ea)

    h = h0.reshape(K * N_PAD, HID)
    for _ in range(T - 1):
        parts1 = _s3_msg(h, src, dst, norm2d)
        h = _t2_call(parts1, rootb, conv1_w).reshape(K * N_PAD, HID)
    parts1 = _s3_msg(h, src, dst, norm2d)
    root2b, h20, c1 = _t2b_call(
        parts1, rootb, bn1_gamma.reshape(1, HID), bn1_beta.reshape(1, HID),
        bn1_mean.reshape(1, HID), bn1_var.reshape(1, HID),
        conv2_root_w, conv2_init_w, conv2_bias, conv2_w)

    w2row = jnp.broadcast_to(
        conv2_w.reshape(K, 1).astype(F32), (K, L)).reshape(K * L)
    zeros2 = jnp.zeros((NC, K * N_PAD), F32)
    cvec = h20.reshape(K * N_PAD)
    c1f = c1.reshape(K * N_PAD)
    parts2 = _s4_msg(zeros2, cvec, w2row, src, dst, norm2d)
    for _ in range(T - 1):
        parts2 = _s4_msg(parts2, c1f, w2row, src, dst, norm2d)

    out = _f1_call(parts2.reshape(NC, K, N_PAD), root2b.reshape(K, N_PAD))
    return out[:N].reshape(N, 1)


# speedup vs baseline: 202.2026x; 1.5265x over previous
import functools

import jax
import jax.numpy as jnp
from jax import lax
from jax.experimental import pallas as pl
from jax.experimental.pallas import tpu as pltpu
from jax.experimental.pallas import tpu_sc as plsc

N = 100000
E = 1600000
F_IN = 128
HID = 16
K = 3
T = 4

NC, NS, L = 2, 16, 16
NW = NC * NS
N_PAD = 100352
NPT = N_PAD // NS
E_PAD = 1605632
E2 = E_PAD // 128
EPT = E_PAD // NW
MROWS = 56
MEDG = MROWS * 128
NMAC = EPT // MEDG
MR3 = 8
MEDG3 = MR3 * 128
NMAC3 = EPT // MEDG3
F32 = jnp.float32
I32 = jnp.int32

_sc_mesh = plsc.VectorSubcoreMesh(core_axis_name="c", subcore_axis_name="s")


def _wid():
    return lax.axis_index("s") * NC + lax.axis_index("c")


def _bcast_lane(nv, e):
    return lax.gather(
        nv, jnp.full((L, 1), e, I32),
        lax.GatherDimensionNumbers(
            offset_dims=(), collapsed_slice_dims=(0,), start_index_map=(0,)),
        (1,), mode=lax.GatherScatterMode.PROMISE_IN_BOUNDS)


@functools.partial(
    pl.kernel,
    out_type=jax.ShapeDtypeStruct((NC, N_PAD), F32),
    mesh=_sc_mesh,
    compiler_params=pltpu.CompilerParams(use_tc_tiling_on_sc=False),
    scratch_types=[
        pltpu.VMEM_SHARED((N_PAD,), F32),
        pltpu.VMEM((MROWS, 128), I32),
        pltpu.VMEM((MROWS, 128), F32),
        pltpu.VMEM((NPT,), F32),
        pltpu.SemaphoreType.DMA,
    ],
)
def _s1_deg(dst2d, ea2d, deg_out, sp_deg, idxb, valb, zbuf, sem):
    cid = lax.axis_index("c")
    sid = lax.axis_index("s")
    wid = _wid()

    def zb(i, _):
        zbuf[pl.ds(i * L, L)] = jnp.zeros((L,), F32)
        return 0
    lax.fori_loop(0, NPT // L, zb, 0)
    pltpu.sync_copy(zbuf, sp_deg.at[pl.ds(sid * NPT, NPT)])
    plsc.subcore_barrier()

    row0 = wid * (EPT // 128)

    def macro(m, _):
        base = row0 + m * MROWS
        d1 = pltpu.async_copy(dst2d.at[pl.ds(base, MROWS)], idxb, sem)
        d2 = pltpu.async_copy(ea2d.at[pl.ds(base, MROWS)], valb, sem)
        d1.wait()
        d2.wait()
        descs = []
        for j in range(MROWS):
            descs.append(pltpu.async_copy(
                valb.at[j], sp_deg.at[idxb.at[j]], sem, add=True))
        for d in descs:
            d.wait()
        return 0
    lax.fori_loop(0, NMAC, macro, 0)
    plsc.subcore_barrier()
    pltpu.sync_copy(sp_deg.at[pl.ds(sid * NPT, NPT)],
                    deg_out.at[cid, pl.ds(sid * NPT, NPT)])


@functools.partial(
    pl.kernel,
    out_type=jax.ShapeDtypeStruct((E2, 128), F32),
    mesh=_sc_mesh,
    compiler_params=pltpu.CompilerParams(use_tc_tiling_on_sc=False),
    scratch_types=[
        pltpu.VMEM_SHARED((N_PAD,), F32),
        pltpu.VMEM((MROWS, 128), I32),
        pltpu.VMEM((MROWS, 128), I32),
        pltpu.VMEM((MROWS, 128), F32),
        pltpu.VMEM((MROWS, 128), F32),
        pltpu.VMEM((MROWS, 128), F32),
        pltpu.SemaphoreType.DMA,
    ],
)
def _s2_norm(dis_tab, src2d, dst2d, ea2d, norm_out,
             sp_dis, sidx, didx, eab, gsb, gdb, sem):
    sid = lax.axis_index("s")
    wid = _wid()

    pltpu.sync_copy(dis_tab.at[pl.ds(sid * NPT, NPT)],
                    sp_dis.at[pl.ds(sid * NPT, NPT)])
    plsc.subcore_barrier()

    row0 = wid * (EPT // 128)

    def macro(m, _):
        base = row0 + m * MROWS
        c1 = pltpu.async_copy(src2d.at[pl.ds(base, MROWS)], sidx, sem)
        c2 = pltpu.async_copy(dst2d.at[pl.ds(base, MROWS)], didx, sem)
        c3 = pltpu.async_copy(ea2d.at[pl.ds(base, MROWS)], eab, sem)
        c1.wait()
        c2.wait()
        c3.wait()
        descs = []
        for j in range(MROWS):
            descs.append(pltpu.async_copy(sp_dis.at[sidx.at[j]], gsb.at[j], sem))
            descs.append(pltpu.async_copy(sp_dis.at[didx.at[j]], gdb.at[j], sem))
        for d in descs:
            d.wait()

        def mul(r, _):
            for c in range(8):
                s = pl.ds(c * L, L)
                eab[r, s] = gsb[r, s] * eab[r, s] * gdb[r, s]
            return 0
        lax.fori_loop(0, MROWS, mul, 0)
        pltpu.sync_copy(eab, norm_out.at[pl.ds(base, MROWS)])
        return 0
    lax.fori_loop(0, NMAC, macro, 0)


@functools.partial(
    pl.kernel,
    out_type=jax.ShapeDtypeStruct((NC, K, N_PAD, HID), F32),
    mesh=_sc_mesh,
    compiler_params=pltpu.CompilerParams(use_tc_tiling_on_sc=False),
    scratch_types=[
        pltpu.VMEM_SHARED((N_PAD, HID), F32),
        pltpu.VMEM((MR3, 128), I32),
        pltpu.VMEM((MR3, 128), I32),
        pltpu.VMEM((MR3, 128), I32),
        pltpu.VMEM((MR3, 128), F32),
        pltpu.VMEM((MEDG3, HID), F32),
        pltpu.VMEM((NPT // 16, HID), F32),
        pltpu.SemaphoreType.DMA,
        pltpu.SemaphoreType.DMA,
    ],
)
def _s3_msg(h_tab, src2d, dst2d, norm2d, parts,
            sp_agg, sidx, gidx, didx, nrm, rows, zbuf, sem, sem2):
    cid = lax.axis_index("c")
    sid = lax.axis_index("s")
    wid = _wid()
    row0 = wid * (EPT // 128)

    def zb(i, _):
        zbuf[i, :] = jnp.zeros((L,), F32)
        return 0
    lax.fori_loop(0, NPT // 16, zb, 0)

    for k in range(K):
        koff = jnp.int32(k * N_PAD)
        for z in range(16):
            pltpu.sync_copy(
                zbuf, sp_agg.at[pl.ds(sid * NPT + z * (NPT // 16), NPT // 16)])
        plsc.subcore_barrier()

        def macro(m, _):
            base = row0 + m * MR3
            c1 = pltpu.async_copy(src2d.at[pl.ds(base, MR3)], sidx, sem)
            c2 = pltpu.async_copy(dst2d.at[pl.ds(base, MR3)], didx, sem)
            c3 = pltpu.async_copy(norm2d.at[pl.ds(base, MR3)], nrm, sem)
            c1.wait()

            def addk(r, _):
                for c in range(8):
                    s = pl.ds(c * L, L)
                    gidx[r, s] = sidx[r, s] + koff
                return 0
            lax.fori_loop(0, MR3, addk, 0)
            gd = []
            for j in range(MR3):
                gd.append(pltpu.async_copy(
                    h_tab.at[gidx.at[j]], rows.at[pl.ds(j * 128, 128)], sem2))
            c2.wait()
            c3.wait()
            for d in gd:
                d.wait()

            def scale(r, _):
                for c in range(8):
                    nv = nrm[r, pl.ds(c * L, L)]
                    for e in range(L):
                        i = r * 128 + c * L + e
                        rows[i, :] = rows[i, :] * _bcast_lane(nv, e)
                return 0
            lax.fori_loop(0, MR3, scale, 0)

            sc = []
            for j in range(MR3):
                sc.append(pltpu.async_copy(
                    rows.at[pl.ds(j * 128, 128)], sp_agg.at[didx.at[j]],
                    sem2, add=True))
            for d in sc:
                d.wait()
            return 0
        lax.fori_loop(0, NMAC3, macro, 0)
        plsc.subcore_barrier()
        pltpu.sync_copy(sp_agg.at[pl.ds(sid * NPT, NPT)],
                        parts.at[cid, k, pl.ds(sid * NPT, NPT)])
        plsc.subcore_barrier()


@functools.partial(
    pl.kernel,
    out_type=jax.ShapeDtypeStruct((NC, K * N_PAD), F32),
    mesh=_sc_mesh,
    compiler_params=pltpu.CompilerParams(use_tc_tiling_on_sc=False),
    scratch_types=[
        pltpu.VMEM_SHARED((K * N_PAD,), F32),
        pltpu.VMEM_SHARED((K * N_PAD,), F32),
        pltpu.VMEM((MROWS, 128), I32),
        pltpu.VMEM((MROWS, 128), I32),
        pltpu.VMEM((MROWS, 128), I32),
        pltpu.VMEM((MROWS, 128), F32),
        pltpu.VMEM((MROWS, 128), F32),
        pltpu.VMEM((NPT,), F32),
        pltpu.VMEM((NPT,), F32),
        pltpu.VMEM((NPT,), F32),
        pltpu.VMEM((L,), F32),
        pltpu.SemaphoreType.DMA,
        pltpu.SemaphoreType.DMA,
    ],
)
def _s4_msg(pprev, cvec, w2row, src2d, dst2d, norm2d, parts,
            sp_h2, sp_agg, sidx, gidx, didx, nrm, vals,
            p0b, p1b, cb, wb, sem, sem2):
    cid = lax.axis_index("c")
    sid = lax.axis_index("s")
    wid = _wid()
    row0 = wid * (EPT // 128)

    def zb(i, _):
        p0b[pl.ds(i * L, L)] = jnp.zeros((L,), F32)
        return 0
    for k in range(K):
        off = k * N_PAD + sid * NPT
        pltpu.sync_copy(w2row.at[pl.ds(k * L, L)], wb)
        d1 = pltpu.async_copy(pprev.at[0, pl.ds(off, NPT)], p0b, sem)
        d2 = pltpu.async_copy(pprev.at[1, pl.ds(off, NPT)], p1b, sem)
        d3 = pltpu.async_copy(cvec.at[pl.ds(off, NPT)], cb, sem)
        d1.wait()
        d2.wait()
        d3.wait()
        wv = wb[pl.ds(0, L)]

        def mk(i, _):
            s = pl.ds(i * L, L)
            cb[s] = wv * (p0b[s] + p1b[s]) + cb[s]
            return 0
        lax.fori_loop(0, NPT // L, mk, 0)
        pltpu.sync_copy(cb, sp_h2.at[pl.ds(off, NPT)])
        lax.fori_loop(0, NPT // L, zb, 0)
        pltpu.sync_copy(p0b, sp_agg.at[pl.ds(off, NPT)])
    plsc.subcore_barrier()

    def macro(m, _):
        base = row0 + m * MROWS
        c1 = pltpu.async_copy(src2d.at[pl.ds(base, MROWS)], sidx, sem)
        c2 = pltpu.async_copy(dst2d.at[pl.ds(base, MROWS)], didx, sem)
        c3 = pltpu.async_copy(norm2d.at[pl.ds(base, MROWS)], nrm, sem)
        c1.wait()
        c2.wait()
        c3.wait()
        for k in range(K):
            koff = jnp.int32(k * N_PAD)

            def addk(r, _):
                for c in range(8):
                    s = pl.ds(c * L, L)
                    gidx[r, s] = sidx[r, s] + koff
                return 0
            lax.fori_loop(0, MROWS, addk, 0)
            gd = []
            for j in range(MROWS):
                gd.append(pltpu.async_copy(
                    sp_h2.at[gidx.at[j]], vals.at[j], sem2))
            for d in gd:
                d.wait()

            def mul(r, _):
                for c in range(8):
                    s = pl.ds(c * L, L)
                    vals[r, s] = vals[r, s] * nrm[r, s]
                    gidx[r, s] = didx[r, s] + koff
                return 0
            lax.fori_loop(0, MROWS, mul, 0)
            sc = []
            for j in range(MROWS):
                sc.append(pltpu.async_copy(
                    vals.at[j], sp_agg.at[gidx.at[j]], sem2, add=True))
            for d in sc:
                d.wait()
        return 0
    lax.fori_loop(0, NMAC, macro, 0)
    plsc.subcore_barrier()
    for k in range(K):
        off = k * N_PAD + sid * NPT
        pltpu.sync_copy(sp_agg.at[pl.ds(off, NPT)],
                        parts.at[cid, pl.ds(off, NPT)])


NP8 = N_PAD // 8
_T1B = 512
_NBLK1 = N_PAD // _T1B
_T2B = 1792
_NBLK2 = NP8 // _T2B


def _t1_body(x_ref, rw_ref, iw_ref, b_ref, deg_ref, root_ref, h0_ref,
             dis_ref):
    xb = x_ref[...]
    for k in range(K):
        root_ref[k] = jnp.dot(xb, rw_ref[k], preferred_element_type=F32) \
            + b_ref[k]
        h0_ref[k] = jnp.dot(xb, iw_ref[k], preferred_element_type=F32)
    d = deg_ref[0] + deg_ref[1]
    dis_ref[...] = jnp.where(d > 0.0, lax.rsqrt(jnp.abs(d) + 1e-30), 0.0)


def _t1_call(x, rw, iw, b, deg_parts):
    return pl.pallas_call(
        _t1_body,
        grid=(_NBLK1,),
        in_specs=[
            pl.BlockSpec((_T1B, F_IN), lambda i: (i, 0)),
            pl.BlockSpec((K, F_IN, HID), lambda i: (0, 0, 0)),
            pl.BlockSpec((K, F_IN, HID), lambda i: (0, 0, 0)),
            pl.BlockSpec((K, 1, HID), lambda i: (0, 0, 0)),
            pl.BlockSpec((NC, _T1B), lambda i: (0, i)),
        ],
        out_specs=[
            pl.BlockSpec((K, _T1B, HID), lambda i: (0, i, 0)),
            pl.BlockSpec((K, _T1B, HID), lambda i: (0, i, 0)),
            pl.BlockSpec((_T1B,), lambda i: (i,)),
        ],
        out_shape=[
            jax.ShapeDtypeStruct((K, N_PAD, HID), F32),
            jax.ShapeDtypeStruct((K, N_PAD, HID), F32),
            jax.ShapeDtypeStruct((N_PAD,), F32),
        ],
    )(x, rw, iw, b, deg_parts)


def _t2_body(p_ref, root_ref, w_ref, h_ref):
    for k in range(K):
        out = jnp.maximum(p_ref[0, k] + p_ref[1, k] + root_ref[k], 0.0)
        h_ref[k] = jnp.dot(out, w_ref[k], preferred_element_type=F32)


def _t2_call(parts, rootb, w128):
    return pl.pallas_call(
        _t2_body,
        grid=(_NBLK2,),
        in_specs=[
            pl.BlockSpec((NC, K, _T2B, 128), lambda i: (0, 0, i, 0)),
            pl.BlockSpec((K, _T2B, 128), lambda i: (0, i, 0)),
            pl.BlockSpec((K, 128, 128), lambda i: (0, 0, 0)),
        ],
        out_specs=pl.BlockSpec((K, _T2B, 128), lambda i: (0, i, 0)),
        out_shape=jax.ShapeDtypeStruct((K, NP8, 128), F32),
    )(parts, rootb, w128)


def _t2b_body(p_ref, root_ref, sc_ref, sh_ref, rw2_ref, iw2_ref, b2_ref,
              w2_ref, root2_ref, h20_ref, c1_ref):
    acc = jnp.zeros((_T2B, 128), F32)
    for k in range(K):
        acc = acc + jnp.maximum(p_ref[0, k] + p_ref[1, k] + root_ref[k], 0.0)
    hm = acc * (1.0 / K)
    hbn = jnp.maximum(hm * sc_ref[...] + sh_ref[...], 0.0)
    for k in range(K):
        r2 = jnp.dot(hbn, rw2_ref[k], preferred_element_type=F32) \
            + b2_ref[k, 0, 0]
        h2 = jnp.dot(hbn, iw2_ref[k], preferred_element_type=F32)
        root2_ref[k] = r2
        h20_ref[k] = h2
        c1_ref[k] = r2 * w2_ref[k, 0, 0]


def _t2b_call(parts, rootb, bn_scale, bn_shift, rw2b, iw2b, b2, w2):
    return pl.pallas_call(
        _t2b_body,
        grid=(_NBLK2,),
        in_specs=[
            pl.BlockSpec((NC, K, _T2B, 128), lambda i: (0, 0, i, 0)),
            pl.BlockSpec((K, _T2B, 128), lambda i: (0, i, 0)),
            pl.BlockSpec((1, 128), lambda i: (0, 0)),
            pl.BlockSpec((1, 128), lambda i: (0, 0)),
            pl.BlockSpec((K, 128, 8), lambda i: (0, 0, 0)),
            pl.BlockSpec((K, 128, 8), lambda i: (0, 0, 0)),
            pl.BlockSpec((K, 1, 1), lambda i: (0, 0, 0)),
            pl.BlockSpec((K, 1, 1), lambda i: (0, 0, 0)),
        ],
        out_specs=[
            pl.BlockSpec((K, _T2B, 8), lambda i: (0, i, 0)),
            pl.BlockSpec((K, _T2B, 8), lambda i: (0, i, 0)),
            pl.BlockSpec((K, _T2B, 8), lambda i: (0, i, 0)),
        ],
        out_shape=[
            jax.ShapeDtypeStruct((K, NP8, 8), F32),
            jax.ShapeDtypeStruct((K, NP8, 8), F32),
            jax.ShapeDtypeStruct((K, NP8, 8), F32),
        ],
    )(parts, rootb, bn_scale, bn_shift, rw2b, iw2b, b2, w2)


_F1B = 512


def _f1_body(p_ref, root2_ref, o_ref):
    s = jnp.zeros((_F1B,), F32)
    for k in range(K):
        s = s + p_ref[0, k] + p_ref[1, k] + root2_ref[k]
    o_ref[...] = jax.nn.sigmoid(s * (1.0 / K))


def _f1_call(parts, root2b):
    return pl.pallas_call(
        _f1_body,
        grid=(N_PAD // _F1B,),
        in_specs=[
            pl.BlockSpec((NC, K, _F1B), lambda i: (0, 0, i)),
            pl.BlockSpec((K, _F1B), lambda i: (0, i)),
        ],
        out_specs=pl.BlockSpec((_F1B,), lambda i: (i,)),
        out_shape=jax.ShapeDtypeStruct((N_PAD,), F32),
    )(parts, root2b)


def kernel(x, edge_index, edge_attr, batch,
           conv1_init_w, conv1_w, conv1_root_w, conv1_bias,
           bn1_gamma, bn1_beta, bn1_mean, bn1_var,
           conv2_init_w, conv2_w, conv2_root_w, conv2_bias):
    del batch
    pad = E_PAD - E
    fill = (jnp.arange(pad, dtype=I32) * 37) % N
    src = jnp.concatenate([edge_index[0].astype(I32), fill]).reshape(E2, 128)
    dst = jnp.concatenate([edge_index[1].astype(I32), fill]).reshape(E2, 128)
    ea = jnp.concatenate([edge_attr.astype(F32),
                          jnp.zeros((pad,), F32)]).reshape(E2, 128)

    deg_parts = _s1_deg(dst, ea)
    rootb, h0, dis = _t1_call(x, conv1_root_w, conv1_init_w, conv1_bias,
                              deg_parts)
    norm2d = _s2_norm(dis, src, dst, ea)

    eye8 = jnp.eye(8, dtype=F32)
    w128 = jnp.einsum("ab,kij->kaibj", eye8, conv1_w).reshape(K, 128, 128)
    rw2b = jnp.einsum("ab,kij->kaibj", eye8, conv2_root_w).reshape(K, 128, 8)
    iw2b = jnp.einsum("ab,kij->kaibj", eye8, conv2_init_w).reshape(K, 128, 8)
    bn_scale16 = bn1_gamma * lax.rsqrt(bn1_var + 1e-5)
    bn_shift16 = bn1_beta - bn1_mean * bn_scale16
    bn_scale = jnp.tile(bn_scale16, 8).reshape(1, 128)
    bn_shift = jnp.tile(bn_shift16, 8).reshape(1, 128)

    rootb_p = rootb.reshape(K, NP8, 128)
    h = h0.reshape(K * N_PAD, HID)
    for _ in range(T - 1):
        parts1 = _s3_msg(h, src, dst, norm2d)
        h = _t2_call(parts1.reshape(NC, K, NP8, 128), rootb_p,
                     w128).reshape(K * N_PAD, HID)
    parts1 = _s3_msg(h, src, dst, norm2d)
    root2b, h20, c1 = _t2b_call(
        parts1.reshape(NC, K, NP8, 128), rootb_p, bn_scale, bn_shift,
        rw2b, iw2b, conv2_bias, conv2_w)

    w2row = jnp.broadcast_to(
        conv2_w.reshape(K, 1).astype(F32), (K, L)).reshape(K * L)
    zeros2 = jnp.zeros((NC, K * N_PAD), F32)
    cvec = h20.reshape(K * N_PAD)
    c1f = c1.reshape(K * N_PAD)
    parts2 = _s4_msg(zeros2, cvec, w2row, src, dst, norm2d)
    for _ in range(T - 1):
        parts2 = _s4_msg(parts2, c1f, w2row, src, dst, norm2d)

    out = _f1_call(parts2.reshape(NC, K, N_PAD), root2b.reshape(K, N_PAD))
    return out[:N].reshape(N, 1)


# speedup vs baseline: 264.4547x; 1.3079x over previous
import functools

import jax
import jax.numpy as jnp
from jax import lax
from jax.experimental import pallas as pl
from jax.experimental.pallas import tpu as pltpu
from jax.experimental.pallas import tpu_sc as plsc

N = 100000
E = 1600000
F_IN = 128
HID = 16
K = 3
T = 4

NC, NS, L = 2, 16, 16
NW = NC * NS
N_PAD = 100352
NPT = N_PAD // NS
E_PAD = 1605632
E2 = E_PAD // 128
EPT = E_PAD // NW
MROWS = 56
MEDG = MROWS * 128
NMAC = EPT // MEDG
F32 = jnp.float32
I32 = jnp.int32

_sc_mesh = plsc.VectorSubcoreMesh(core_axis_name="c", subcore_axis_name="s")


def _wid():
    return lax.axis_index("s") * NC + lax.axis_index("c")


def _bcast_lane(nv, e):
    return lax.gather(
        nv, jnp.full((L, 1), e, I32),
        lax.GatherDimensionNumbers(
            offset_dims=(), collapsed_slice_dims=(0,), start_index_map=(0,)),
        (1,), mode=lax.GatherScatterMode.PROMISE_IN_BOUNDS)


@functools.partial(
    pl.kernel,
    out_type=jax.ShapeDtypeStruct((NC, N_PAD), F32),
    mesh=_sc_mesh,
    compiler_params=pltpu.CompilerParams(use_tc_tiling_on_sc=False),
    scratch_types=[
        pltpu.VMEM_SHARED((N_PAD,), F32),
        pltpu.VMEM((MROWS, 128), I32),
        pltpu.VMEM((MROWS, 128), F32),
        pltpu.VMEM((NPT,), F32),
        pltpu.SemaphoreType.DMA,
    ],
)
def _s1_deg(dst2d, ea2d, deg_out, sp_deg, idxb, valb, zbuf, sem):
    cid = lax.axis_index("c")
    sid = lax.axis_index("s")
    wid = _wid()

    def zb(i, _):
        zbuf[pl.ds(i * L, L)] = jnp.zeros((L,), F32)
        return 0
    lax.fori_loop(0, NPT // L, zb, 0)
    pltpu.sync_copy(zbuf, sp_deg.at[pl.ds(sid * NPT, NPT)])
    plsc.subcore_barrier()

    row0 = wid * (EPT // 128)

    def macro(m, _):
        base = row0 + m * MROWS
        d1 = pltpu.async_copy(dst2d.at[pl.ds(base, MROWS)], idxb, sem)
        d2 = pltpu.async_copy(ea2d.at[pl.ds(base, MROWS)], valb, sem)
        d1.wait()
        d2.wait()
        descs = []
        for j in range(MROWS):
            descs.append(pltpu.async_copy(
                valb.at[j], sp_deg.at[idxb.at[j]], sem, add=True))
        for d in descs:
            d.wait()
        return 0
    lax.fori_loop(0, NMAC, macro, 0)
    plsc.subcore_barrier()
    pltpu.sync_copy(sp_deg.at[pl.ds(sid * NPT, NPT)],
                    deg_out.at[cid, pl.ds(sid * NPT, NPT)])


@functools.partial(
    pl.kernel,
    out_type=jax.ShapeDtypeStruct((E2, 128), F32),
    mesh=_sc_mesh,
    compiler_params=pltpu.CompilerParams(use_tc_tiling_on_sc=False),
    scratch_types=[
        pltpu.VMEM_SHARED((N_PAD,), F32),
        pltpu.VMEM((MROWS, 128), I32),
        pltpu.VMEM((MROWS, 128), I32),
        pltpu.VMEM((MROWS, 128), F32),
        pltpu.VMEM((MROWS, 128), F32),
        pltpu.VMEM((MROWS, 128), F32),
        pltpu.SemaphoreType.DMA,
    ],
)
def _s2_norm(dis_tab, src2d, dst2d, ea2d, norm_out,
             sp_dis, sidx, didx, eab, gsb, gdb, sem):
    sid = lax.axis_index("s")
    wid = _wid()

    pltpu.sync_copy(dis_tab.at[pl.ds(sid * NPT, NPT)],
                    sp_dis.at[pl.ds(sid * NPT, NPT)])
    plsc.subcore_barrier()

    row0 = wid * (EPT // 128)

    def macro(m, _):
        base = row0 + m * MROWS
        c1 = pltpu.async_copy(src2d.at[pl.ds(base, MROWS)], sidx, sem)
        c2 = pltpu.async_copy(dst2d.at[pl.ds(base, MROWS)], didx, sem)
        c3 = pltpu.async_copy(ea2d.at[pl.ds(base, MROWS)], eab, sem)
        c1.wait()
        c2.wait()
        c3.wait()
        descs = []
        for j in range(MROWS):
            descs.append(pltpu.async_copy(sp_dis.at[sidx.at[j]], gsb.at[j], sem))
            descs.append(pltpu.async_copy(sp_dis.at[didx.at[j]], gdb.at[j], sem))
        for d in descs:
            d.wait()

        def mul(r, _):
            for c in range(8):
                s = pl.ds(c * L, L)
                eab[r, s] = gsb[r, s] * eab[r, s] * gdb[r, s]
            return 0
        lax.fori_loop(0, MROWS, mul, 0)
        pltpu.sync_copy(eab, norm_out.at[pl.ds(base, MROWS)])
        return 0
    lax.fori_loop(0, NMAC, macro, 0)


MR3 = 4
MEDG3 = MR3 * 128
NM3 = EPT // MEDG3


@functools.partial(
    pl.kernel,
    out_type=jax.ShapeDtypeStruct((NC, K, N_PAD, HID), F32),
    mesh=_sc_mesh,
    compiler_params=pltpu.CompilerParams(use_tc_tiling_on_sc=False),
    scratch_types=[
        pltpu.VMEM_SHARED((N_PAD, HID), F32),
        pltpu.VMEM((2, MR3, 128), I32),
        pltpu.VMEM((2, MR3, 128), I32),
        pltpu.VMEM((4, MR3, 128), I32),
        pltpu.VMEM((2, MR3, 128), F32),
        pltpu.VMEM((2, MEDG3, HID), F32),
        pltpu.VMEM((NPT // 16, HID), F32),
        pltpu.SemaphoreType.DMA,
        pltpu.SemaphoreType.DMA,
        pltpu.SemaphoreType.DMA,
        pltpu.SemaphoreType.DMA,
    ],
)
def _s3_msg(h_tab, src2d, dst2d, norm2d, parts,
            sp_agg, sidx, gidx, didx, nrm, rows, zbuf, semL, semG, semS,
            semZ):
    cid = lax.axis_index("c")
    sid = lax.axis_index("s")
    wid = _wid()
    row0 = wid * (EPT // 128)

    def zb(i, _):
        zbuf[i, :] = jnp.zeros((L,), F32)
        return 0
    lax.fori_loop(0, NPT // 16, zb, 0)

    def fire_l(m, lp, dp):
        base = row0 + m * MR3
        pltpu.async_copy(src2d.at[pl.ds(base, MR3)], sidx.at[lp], semL)
        pltpu.async_copy(dst2d.at[pl.ds(base, MR3)], didx.at[dp], semL)
        pltpu.async_copy(norm2d.at[pl.ds(base, MR3)], nrm.at[lp], semL)

    def wait_l():
        for _ in range(3):
            pltpu.make_async_copy(
                src2d.at[pl.ds(0, MR3)], sidx.at[0], semL).wait()

    def addk(lp, koff):
        def body(r, _):
            for c in range(8):
                s = pl.ds(c * L, L)
                gidx[lp, r, s] = sidx[lp, r, s] + koff
            return 0
        lax.fori_loop(0, MR3, body, 0)

    def fire_g(lp):
        for j in range(MR3):
            pltpu.async_copy(h_tab.at[gidx.at[lp].at[j]],
                             rows.at[lp].at[pl.ds(j * 128, 128)], semG)

    def drain_g(lp):
        for j in range(MR3):
            pltpu.make_async_copy(h_tab.at[gidx.at[lp].at[j]],
                                  rows.at[lp].at[pl.ds(j * 128, 128)],
                                  semG).wait()

    def scale(lp):
        def body(g, _):
            nv = nrm[lp, g // 8, pl.ds((g % 8) * L, L)]
            for e in range(L):
                i = g * L + e
                rows[lp, i, :] = rows[lp, i, :] * _bcast_lane(nv, e)
            return 0
        lax.fori_loop(0, MEDG3 // L, body, 0)

    def fire_s(lp, dp):
        for j in range(MR3):
            pltpu.async_copy(rows.at[lp].at[pl.ds(j * 128, 128)],
                             sp_agg.at[didx.at[dp].at[j]], semS, add=True)

    def drain_s(lp, dp):
        for j in range(MR3):
            pltpu.make_async_copy(rows.at[lp].at[pl.ds(j * 128, 128)],
                                  sp_agg.at[didx.at[dp].at[j]],
                                  semS).wait()

    for k in range(K):
        koff = jnp.int32(k * N_PAD)
        for z in range(16):
            pltpu.async_copy(
                zbuf, sp_agg.at[pl.ds(sid * NPT + z * (NPT // 16),
                                      NPT // 16)], semZ)
        for z in range(16):
            pltpu.make_async_copy(
                zbuf, sp_agg.at[pl.ds(0, NPT // 16)], semZ).wait()
        plsc.subcore_barrier()

        fire_l(0, 0, 0)
        wait_l()
        addk(0, koff)
        fire_g(0)
        fire_l(1, 1, 1)
        drain_g(0)
        scale(0)
        fire_s(0, 0)
        wait_l()
        addk(1, koff)
        fire_g(1)
        fire_l(2, 0, 2)

        def quad(t, _):
            for u in range(4):
                m = t * 4 + 1 + u
                cur = (1 + u) % 2
                nxt = (2 + u) % 2
                dcur = (1 + u) % 4
                dnxt2 = (3 + u) % 4
                mm = t * 4 + (1 + u)
                drain_g(cur)
                wait_l()
                addk(nxt, koff)
                drain_s(nxt, (u) % 4)
                fire_g(nxt)
                scale(cur)
                fire_s(cur, dcur)
                base = jnp.minimum(row0 + (mm + 2) * MR3,
                                   jnp.int32(E2 - MR3))
                pltpu.async_copy(src2d.at[pl.ds(base, MR3)],
                                 sidx.at[cur], semL)
                pltpu.async_copy(dst2d.at[pl.ds(base, MR3)],
                                 didx.at[dnxt2], semL)
                pltpu.async_copy(norm2d.at[pl.ds(base, MR3)],
                                 nrm.at[cur], semL)
            return 0
        lax.fori_loop(0, (NM3 - 2) // 4, quad, 0)

        drain_g(1)
        wait_l()
        drain_s(0, 0)
        scale(1)
        fire_s(1, 1)
        drain_s(1, 1)
        plsc.subcore_barrier()
        pltpu.sync_copy(sp_agg.at[pl.ds(sid * NPT, NPT)],
                        parts.at[cid, k, pl.ds(sid * NPT, NPT)])
        plsc.subcore_barrier()


@functools.partial(
    pl.kernel,
    out_type=jax.ShapeDtypeStruct((NC, K * N_PAD), F32),
    mesh=_sc_mesh,
    compiler_params=pltpu.CompilerParams(use_tc_tiling_on_sc=False),
    scratch_types=[
        pltpu.VMEM_SHARED((K * N_PAD,), F32),
        pltpu.VMEM_SHARED((K * N_PAD,), F32),
        pltpu.VMEM((MROWS, 128), I32),
        pltpu.VMEM((MROWS, 128), I32),
        pltpu.VMEM((MROWS, 128), I32),
        pltpu.VMEM((MROWS, 128), F32),
        pltpu.VMEM((MROWS, 128), F32),
        pltpu.VMEM((NPT,), F32),
        pltpu.VMEM((NPT,), F32),
        pltpu.VMEM((NPT,), F32),
        pltpu.VMEM((L,), F32),
        pltpu.SemaphoreType.DMA,
        pltpu.SemaphoreType.DMA,
    ],
)
def _s4_msg(pprev, cvec, w2row, src2d, dst2d, norm2d, parts,
            sp_h2, sp_agg, sidx, gidx, didx, nrm, vals,
            p0b, p1b, cb, wb, sem, sem2):
    cid = lax.axis_index("c")
    sid = lax.axis_index("s")
    wid = _wid()
    row0 = wid * (EPT // 128)

    def zb(i, _):
        p0b[pl.ds(i * L, L)] = jnp.zeros((L,), F32)
        return 0
    for k in range(K):
        off = k * N_PAD + sid * NPT
        pltpu.sync_copy(w2row.at[pl.ds(k * L, L)], wb)
        d1 = pltpu.async_copy(pprev.at[0, pl.ds(off, NPT)], p0b, sem)
        d2 = pltpu.async_copy(pprev.at[1, pl.ds(off, NPT)], p1b, sem)
        d3 = pltpu.async_copy(cvec.at[pl.ds(off, NPT)], cb, sem)
        d1.wait()
        d2.wait()
        d3.wait()
        wv = wb[pl.ds(0, L)]

        def mk(i, _):
            s = pl.ds(i * L, L)
            cb[s] = wv * (p0b[s] + p1b[s]) + cb[s]
            return 0
        lax.fori_loop(0, NPT // L, mk, 0)
        pltpu.sync_copy(cb, sp_h2.at[pl.ds(off, NPT)])
        lax.fori_loop(0, NPT // L, zb, 0)
        pltpu.sync_copy(p0b, sp_agg.at[pl.ds(off, NPT)])
    plsc.subcore_barrier()

    def macro(m, _):
        base = row0 + m * MROWS
        c1 = pltpu.async_copy(src2d.at[pl.ds(base, MROWS)], sidx, sem)
        c2 = pltpu.async_copy(dst2d.at[pl.ds(base, MROWS)], didx, sem)
        c3 = pltpu.async_copy(norm2d.at[pl.ds(base, MROWS)], nrm, sem)
        c1.wait()
        c2.wait()
        c3.wait()
        for k in range(K):
            koff = jnp.int32(k * N_PAD)

            def addk(r, _):
                for c in range(8):
                    s = pl.ds(c * L, L)
                    gidx[r, s] = sidx[r, s] + koff
                return 0
            lax.fori_loop(0, MROWS, addk, 0)
            gd = []
            for j in range(MROWS):
                gd.append(pltpu.async_copy(
                    sp_h2.at[gidx.at[j]], vals.at[j], sem2))
            for d in gd:
                d.wait()

            def mul(r, _):
                for c in range(8):
                    s = pl.ds(c * L, L)
                    vals[r, s] = vals[r, s] * nrm[r, s]
                    gidx[r, s] = didx[r, s] + koff
                return 0
            lax.fori_loop(0, MROWS, mul, 0)
            sc = []
            for j in range(MROWS):
                sc.append(pltpu.async_copy(
                    vals.at[j], sp_agg.at[gidx.at[j]], sem2, add=True))
            for d in sc:
                d.wait()
        return 0
    lax.fori_loop(0, NMAC, macro, 0)
    plsc.subcore_barrier()
    for k in range(K):
        off = k * N_PAD + sid * NPT
        pltpu.sync_copy(sp_agg.at[pl.ds(off, NPT)],
                        parts.at[cid, pl.ds(off, NPT)])


NP8 = N_PAD // 8
_T1B = 512
_NBLK1 = N_PAD // _T1B
_T2B = 1792
_NBLK2 = NP8 // _T2B


def _t1_body(x_ref, rw_ref, iw_ref, b_ref, deg_ref, root_ref, h0_ref,
             dis_ref):
    xb = x_ref[...]
    for k in range(K):
        root_ref[k] = jnp.dot(xb, rw_ref[k], preferred_element_type=F32) \
            + b_ref[k]
        h0_ref[k] = jnp.dot(xb, iw_ref[k], preferred_element_type=F32)
    d = deg_ref[0] + deg_ref[1]
    dis_ref[...] = jnp.where(d > 0.0, lax.rsqrt(jnp.abs(d) + 1e-30), 0.0)


def _t1_call(x, rw, iw, b, deg_parts):
    return pl.pallas_call(
        _t1_body,
        grid=(_NBLK1,),
        in_specs=[
            pl.BlockSpec((_T1B, F_IN), lambda i: (i, 0)),
            pl.BlockSpec((K, F_IN, HID), lambda i: (0, 0, 0)),
            pl.BlockSpec((K, F_IN, HID), lambda i: (0, 0, 0)),
            pl.BlockSpec((K, 1, HID), lambda i: (0, 0, 0)),
            pl.BlockSpec((NC, _T1B), lambda i: (0, i)),
        ],
        out_specs=[
            pl.BlockSpec((K, _T1B, HID), lambda i: (0, i, 0)),
            pl.BlockSpec((K, _T1B, HID), lambda i: (0, i, 0)),
            pl.BlockSpec((_T1B,), lambda i: (i,)),
        ],
        out_shape=[
            jax.ShapeDtypeStruct((K, N_PAD, HID), F32),
            jax.ShapeDtypeStruct((K, N_PAD, HID), F32),
            jax.ShapeDtypeStruct((N_PAD,), F32),
        ],
    )(x, rw, iw, b, deg_parts)


def _t2_body(p_ref, root_ref, w_ref, h_ref):
    for k in range(K):
        out = jnp.maximum(p_ref[0, k] + p_ref[1, k] + root_ref[k], 0.0)
        h_ref[k] = jnp.dot(out, w_ref[k], preferred_element_type=F32)


def _t2_call(parts, rootb, w128):
    return pl.pallas_call(
        _t2_body,
        grid=(_NBLK2,),
        in_specs=[
            pl.BlockSpec((NC, K, _T2B, 128), lambda i: (0, 0, i, 0)),
            pl.BlockSpec((K, _T2B, 128), lambda i: (0, i, 0)),
            pl.BlockSpec((K, 128, 128), lambda i: (0, 0, 0)),
        ],
        out_specs=pl.BlockSpec((K, _T2B, 128), lambda i: (0, i, 0)),
        out_shape=jax.ShapeDtypeStruct((K, NP8, 128), F32),
    )(parts, rootb, w128)


def _t2b_body(p_ref, root_ref, sc_ref, sh_ref, rw2_ref, iw2_ref, b2_ref,
              w2_ref, root2_ref, h20_ref, c1_ref):
    acc = jnp.zeros((_T2B, 128), F32)
    for k in range(K):
        acc = acc + jnp.maximum(p_ref[0, k] + p_ref[1, k] + root_ref[k], 0.0)
    hm = acc * (1.0 / K)
    hbn = jnp.maximum(hm * sc_ref[...] + sh_ref[...], 0.0)
    for k in range(K):
        r2 = jnp.dot(hbn, rw2_ref[k], preferred_element_type=F32) \
            + b2_ref[k, 0, 0]
        h2 = jnp.dot(hbn, iw2_ref[k], preferred_element_type=F32)
        root2_ref[k] = r2
        h20_ref[k] = h2
        c1_ref[k] = r2 * w2_ref[k, 0, 0]


def _t2b_call(parts, rootb, bn_scale, bn_shift, rw2b, iw2b, b2, w2):
    return pl.pallas_call(
        _t2b_body,
        grid=(_NBLK2,),
        in_specs=[
            pl.BlockSpec((NC, K, _T2B, 128), lambda i: (0, 0, i, 0)),
            pl.BlockSpec((K, _T2B, 128), lambda i: (0, i, 0)),
            pl.BlockSpec((1, 128), lambda i: (0, 0)),
            pl.BlockSpec((1, 128), lambda i: (0, 0)),
            pl.BlockSpec((K, 128, 8), lambda i: (0, 0, 0)),
            pl.BlockSpec((K, 128, 8), lambda i: (0, 0, 0)),
            pl.BlockSpec((K, 1, 1), lambda i: (0, 0, 0)),
            pl.BlockSpec((K, 1, 1), lambda i: (0, 0, 0)),
        ],
        out_specs=[
            pl.BlockSpec((K, _T2B, 8), lambda i: (0, i, 0)),
            pl.BlockSpec((K, _T2B, 8), lambda i: (0, i, 0)),
            pl.BlockSpec((K, _T2B, 8), lambda i: (0, i, 0)),
        ],
        out_shape=[
            jax.ShapeDtypeStruct((K, NP8, 8), F32),
            jax.ShapeDtypeStruct((K, NP8, 8), F32),
            jax.ShapeDtypeStruct((K, NP8, 8), F32),
        ],
    )(parts, rootb, bn_scale, bn_shift, rw2b, iw2b, b2, w2)


_F1B = 512


def _f1_body(p_ref, root2_ref, o_ref):
    s = jnp.zeros((_F1B,), F32)
    for k in range(K):
        s = s + p_ref[0, k] + p_ref[1, k] + root2_ref[k]
    o_ref[...] = jax.nn.sigmoid(s * (1.0 / K))


def _f1_call(parts, root2b):
    return pl.pallas_call(
        _f1_body,
        grid=(N_PAD // _F1B,),
        in_specs=[
            pl.BlockSpec((NC, K, _F1B), lambda i: (0, 0, i)),
            pl.BlockSpec((K, _F1B), lambda i: (0, i)),
        ],
        out_specs=pl.BlockSpec((_F1B,), lambda i: (i,)),
        out_shape=jax.ShapeDtypeStruct((N_PAD,), F32),
    )(parts, root2b)


def kernel(x, edge_index, edge_attr, batch,
           conv1_init_w, conv1_w, conv1_root_w, conv1_bias,
           bn1_gamma, bn1_beta, bn1_mean, bn1_var,
           conv2_init_w, conv2_w, conv2_root_w, conv2_bias):
    del batch
    pad = E_PAD - E
    fill = (jnp.arange(pad, dtype=I32) * 37) % N
    src = jnp.concatenate([edge_index[0].astype(I32), fill]).reshape(E2, 128)
    dst = jnp.concatenate([edge_index[1].astype(I32), fill]).reshape(E2, 128)
    ea = jnp.concatenate([edge_attr.astype(F32),
                          jnp.zeros((pad,), F32)]).reshape(E2, 128)

    deg_parts = _s1_deg(dst, ea)
    rootb, h0, dis = _t1_call(x, conv1_root_w, conv1_init_w, conv1_bias,
                              deg_parts)
    norm2d = _s2_norm(dis, src, dst, ea)

    eye8 = jnp.eye(8, dtype=F32)
    w128 = jnp.einsum("ab,kij->kaibj", eye8, conv1_w).reshape(K, 128, 128)
    rw2b = jnp.einsum("ab,kij->kaibj", eye8, conv2_root_w).reshape(K, 128, 8)
    iw2b = jnp.einsum("ab,kij->kaibj", eye8, conv2_init_w).reshape(K, 128, 8)
    bn_scale16 = bn1_gamma * lax.rsqrt(bn1_var + 1e-5)
    bn_shift16 = bn1_beta - bn1_mean * bn_scale16
    bn_scale = jnp.tile(bn_scale16, 8).reshape(1, 128)
    bn_shift = jnp.tile(bn_shift16, 8).reshape(1, 128)

    rootb_p = rootb.reshape(K, NP8, 128)
    h = h0.reshape(K * N_PAD, HID)
    for _ in range(T - 1):
        parts1 = _s3_msg(h, src, dst, norm2d)
        h = _t2_call(parts1.reshape(NC, K, NP8, 128), rootb_p,
                     w128).reshape(K * N_PAD, HID)
    parts1 = _s3_msg(h, src, dst, norm2d)
    root2b, h20, c1 = _t2b_call(
        parts1.reshape(NC, K, NP8, 128), rootb_p, bn_scale, bn_shift,
        rw2b, iw2b, conv2_bias, conv2_w)

    w2row = jnp.broadcast_to(
        conv2_w.reshape(K, 1).astype(F32), (K, L)).reshape(K * L)
    zeros2 = jnp.zeros((NC, K * N_PAD), F32)
    cvec = h20.reshape(K * N_PAD)
    c1f = c1.reshape(K * N_PAD)
    parts2 = _s4_msg(zeros2, cvec, w2row, src, dst, norm2d)
    for _ in range(T - 1):
        parts2 = _s4_msg(parts2, c1f, w2row, src, dst, norm2d)

    out = _f1_call(parts2.reshape(NC, K, N_PAD), root2b.reshape(K, N_PAD))
    return out[:N].reshape(N, 1)


# speedup vs baseline: 295.0379x; 1.1156x over previous
import functools

import jax
import jax.numpy as jnp
from jax import lax
from jax.experimental import pallas as pl
from jax.experimental.pallas import tpu as pltpu
from jax.experimental.pallas import tpu_sc as plsc

N = 100000
E = 1600000
F_IN = 128
HID = 16
K = 3
T = 4

NC, NS, L = 2, 16, 16
NW = NC * NS
N_PAD = 100352
NPT = N_PAD // NS
E_PAD = 1605632
E2 = E_PAD // 128
EPT = E_PAD // NW
MROWS = 56
MEDG = MROWS * 128
NMAC = EPT // MEDG
F32 = jnp.float32
I32 = jnp.int32

_sc_mesh = plsc.VectorSubcoreMesh(core_axis_name="c", subcore_axis_name="s")


def _wid():
    return lax.axis_index("s") * NC + lax.axis_index("c")


def _bcast_lane(nv, e):
    return lax.gather(
        nv, jnp.full((L, 1), e, I32),
        lax.GatherDimensionNumbers(
            offset_dims=(), collapsed_slice_dims=(0,), start_index_map=(0,)),
        (1,), mode=lax.GatherScatterMode.PROMISE_IN_BOUNDS)


@functools.partial(
    pl.kernel,
    out_type=jax.ShapeDtypeStruct((NC, N_PAD), F32),
    mesh=_sc_mesh,
    compiler_params=pltpu.CompilerParams(use_tc_tiling_on_sc=False),
    scratch_types=[
        pltpu.VMEM_SHARED((N_PAD,), F32),
        pltpu.VMEM((MROWS, 128), I32),
        pltpu.VMEM((MROWS, 128), F32),
        pltpu.VMEM((NPT,), F32),
        pltpu.SemaphoreType.DMA,
    ],
)
def _s1_deg(dst2d, ea2d, deg_out, sp_deg, idxb, valb, zbuf, sem):
    cid = lax.axis_index("c")
    sid = lax.axis_index("s")
    wid = _wid()

    def zb(i, _):
        zbuf[pl.ds(i * L, L)] = jnp.zeros((L,), F32)
        return 0
    lax.fori_loop(0, NPT // L, zb, 0)
    pltpu.sync_copy(zbuf, sp_deg.at[pl.ds(sid * NPT, NPT)])
    plsc.subcore_barrier()

    row0 = wid * (EPT // 128)

    def macro(m, _):
        base = row0 + m * MROWS
        d1 = pltpu.async_copy(dst2d.at[pl.ds(base, MROWS)], idxb, sem)
        d2 = pltpu.async_copy(ea2d.at[pl.ds(base, MROWS)], valb, sem)
        d1.wait()
        d2.wait()
        descs = []
        for j in range(MROWS):
            descs.append(pltpu.async_copy(
                valb.at[j], sp_deg.at[idxb.at[j]], sem, add=True))
        for d in descs:
            d.wait()
        return 0
    lax.fori_loop(0, NMAC, macro, 0)
    plsc.subcore_barrier()
    pltpu.sync_copy(sp_deg.at[pl.ds(sid * NPT, NPT)],
                    deg_out.at[cid, pl.ds(sid * NPT, NPT)])


@functools.partial(
    pl.kernel,
    out_type=jax.ShapeDtypeStruct((E2, 128), F32),
    mesh=_sc_mesh,
    compiler_params=pltpu.CompilerParams(use_tc_tiling_on_sc=False),
    scratch_types=[
        pltpu.VMEM_SHARED((N_PAD,), F32),
        pltpu.VMEM((MROWS, 128), I32),
        pltpu.VMEM((MROWS, 128), I32),
        pltpu.VMEM((MROWS, 128), F32),
        pltpu.VMEM((MROWS, 128), F32),
        pltpu.VMEM((MROWS, 128), F32),
        pltpu.SemaphoreType.DMA,
    ],
)
def _s2_norm(dis_tab, src2d, dst2d, ea2d, norm_out,
             sp_dis, sidx, didx, eab, gsb, gdb, sem):
    sid = lax.axis_index("s")
    wid = _wid()

    pltpu.sync_copy(dis_tab.at[pl.ds(sid * NPT, NPT)],
                    sp_dis.at[pl.ds(sid * NPT, NPT)])
    plsc.subcore_barrier()

    row0 = wid * (EPT // 128)

    def macro(m, _):
        base = row0 + m * MROWS
        c1 = pltpu.async_copy(src2d.at[pl.ds(base, MROWS)], sidx, sem)
        c2 = pltpu.async_copy(dst2d.at[pl.ds(base, MROWS)], didx, sem)
        c3 = pltpu.async_copy(ea2d.at[pl.ds(base, MROWS)], eab, sem)
        c1.wait()
        c2.wait()
        c3.wait()
        descs = []
        for j in range(MROWS):
            descs.append(pltpu.async_copy(sp_dis.at[sidx.at[j]], gsb.at[j], sem))
            descs.append(pltpu.async_copy(sp_dis.at[didx.at[j]], gdb.at[j], sem))
        for d in descs:
            d.wait()

        def mul(r, _):
            for c in range(8):
                s = pl.ds(c * L, L)
                eab[r, s] = gsb[r, s] * eab[r, s] * gdb[r, s]
            return 0
        lax.fori_loop(0, MROWS, mul, 0)
        pltpu.sync_copy(eab, norm_out.at[pl.ds(base, MROWS)])
        return 0
    lax.fori_loop(0, NMAC, macro, 0)


MR3 = 4
MEDG3 = MR3 * 128
NM3 = EPT // MEDG3


@functools.partial(
    pl.kernel,
    out_type=jax.ShapeDtypeStruct((NC, K, N_PAD, HID), F32),
    mesh=_sc_mesh,
    compiler_params=pltpu.CompilerParams(use_tc_tiling_on_sc=False),
    scratch_types=[
        pltpu.VMEM_SHARED((N_PAD, HID), F32),
        pltpu.VMEM((2, MR3, 128), I32),
        pltpu.VMEM((2, MR3, 128), I32),
        pltpu.VMEM((4, MR3, 128), I32),
        pltpu.VMEM((2, MR3, 128), F32),
        pltpu.VMEM((2, MEDG3, HID), F32),
        pltpu.VMEM((NPT // 16, HID), F32),
        pltpu.SemaphoreType.DMA,
        pltpu.SemaphoreType.DMA,
        pltpu.SemaphoreType.DMA,
        pltpu.SemaphoreType.DMA,
    ],
)
def _s3_msg(h_tab, src2d, dst2d, norm2d, parts,
            sp_agg, sidx, gidx, didx, nrm, rows, zbuf, semL, semG, semS,
            semZ):
    cid = lax.axis_index("c")
    sid = lax.axis_index("s")
    wid = _wid()
    row0 = wid * (EPT // 128)

    def zb(i, _):
        zbuf[i, :] = jnp.zeros((L,), F32)
        return 0
    lax.fori_loop(0, NPT // 16, zb, 0)

    def fire_l(m, lp, dp):
        base = row0 + m * MR3
        pltpu.async_copy(src2d.at[pl.ds(base, MR3)], sidx.at[lp], semL)
        pltpu.async_copy(dst2d.at[pl.ds(base, MR3)], didx.at[dp], semL)
        pltpu.async_copy(norm2d.at[pl.ds(base, MR3)], nrm.at[lp], semL)

    def wait_l():
        for _ in range(3):
            pltpu.make_async_copy(
                src2d.at[pl.ds(0, MR3)], sidx.at[0], semL).wait()

    def addk(lp, koff):
        def body(r, _):
            for c in range(8):
                s = pl.ds(c * L, L)
                gidx[lp, r, s] = sidx[lp, r, s] + koff
            return 0
        lax.fori_loop(0, MR3, body, 0)

    def fire_g(lp):
        for j in range(MR3):
            pltpu.async_copy(h_tab.at[gidx.at[lp].at[j]],
                             rows.at[lp].at[pl.ds(j * 128, 128)], semG)

    def drain_g(lp):
        for j in range(MR3):
            pltpu.make_async_copy(h_tab.at[gidx.at[lp].at[j]],
                                  rows.at[lp].at[pl.ds(j * 128, 128)],
                                  semG).wait()

    def scale(lp):
        def body(g, _):
            nv = nrm[lp, g // 8, pl.ds((g % 8) * L, L)]
            for e in range(L):
                i = g * L + e
                rows[lp, i, :] = rows[lp, i, :] * _bcast_lane(nv, e)
            return 0
        lax.fori_loop(0, MEDG3 // L, body, 0)

    def fire_s(lp, dp):
        for j in range(MR3):
            pltpu.async_copy(rows.at[lp].at[pl.ds(j * 128, 128)],
                             sp_agg.at[didx.at[dp].at[j]], semS, add=True)

    def drain_s(lp, dp):
        for j in range(MR3):
            pltpu.make_async_copy(rows.at[lp].at[pl.ds(j * 128, 128)],
                                  sp_agg.at[didx.at[dp].at[j]],
                                  semS).wait()

    for k in range(K):
        koff = jnp.int32(k * N_PAD)
        for z in range(16):
            pltpu.async_copy(
                zbuf, sp_agg.at[pl.ds(sid * NPT + z * (NPT // 16),
                                      NPT // 16)], semZ)
        for z in range(16):
            pltpu.make_async_copy(
                zbuf, sp_agg.at[pl.ds(0, NPT // 16)], semZ).wait()
        plsc.subcore_barrier()

        fire_l(0, 0, 0)
        wait_l()
        addk(0, koff)
        fire_g(0)
        fire_l(1, 1, 1)
        drain_g(0)
        scale(0)
        fire_s(0, 0)
        wait_l()
        addk(1, koff)
        fire_g(1)
        fire_l(2, 0, 2)

        def quad(t, _):
            for u in range(4):
                m = t * 4 + 1 + u
                cur = (1 + u) % 2
                nxt = (2 + u) % 2
                dcur = (1 + u) % 4
                dnxt2 = (3 + u) % 4
                mm = t * 4 + (1 + u)
                drain_g(cur)
                wait_l()
                addk(nxt, koff)
                drain_s(nxt, (u) % 4)
                fire_g(nxt)
                scale(cur)
                fire_s(cur, dcur)
                base = jnp.minimum(row0 + (mm + 2) * MR3,
                                   jnp.int32(E2 - MR3))
                pltpu.async_copy(src2d.at[pl.ds(base, MR3)],
                                 sidx.at[cur], semL)
                pltpu.async_copy(dst2d.at[pl.ds(base, MR3)],
                                 didx.at[dnxt2], semL)
                pltpu.async_copy(norm2d.at[pl.ds(base, MR3)],
                                 nrm.at[cur], semL)
            return 0
        lax.fori_loop(0, (NM3 - 2) // 4, quad, 0)

        drain_g(1)
        wait_l()
        drain_s(0, 0)
        scale(1)
        fire_s(1, 1)
        drain_s(1, 1)
        plsc.subcore_barrier()
        pltpu.sync_copy(sp_agg.at[pl.ds(sid * NPT, NPT)],
                        parts.at[cid, k, pl.ds(sid * NPT, NPT)])
        plsc.subcore_barrier()


MR4 = 8
NM4 = (EPT // 128) // MR4


@functools.partial(
    pl.kernel,
    out_type=jax.ShapeDtypeStruct((NC, K * N_PAD), F32),
    mesh=_sc_mesh,
    compiler_params=pltpu.CompilerParams(use_tc_tiling_on_sc=False),
    scratch_types=[
        pltpu.VMEM_SHARED((K * N_PAD,), F32),
        pltpu.VMEM_SHARED((K * N_PAD,), F32),
        pltpu.VMEM((2, MR4, 128), I32),
        pltpu.VMEM((2, MR4, 128), I32),
        pltpu.VMEM((2, MR4, 128), F32),
        pltpu.VMEM((3, MR4, 128), I32),
        pltpu.VMEM((3, MR4, 128), I32),
        pltpu.VMEM((3, MR4, 128), F32),
        pltpu.VMEM((NPT,), F32),
        pltpu.VMEM((NPT,), F32),
        pltpu.VMEM((NPT,), F32),
        pltpu.VMEM((L,), F32),
        pltpu.SemaphoreType.DMA,
        pltpu.SemaphoreType.DMA,
        pltpu.SemaphoreType.DMA,
    ],
)
def _s4_msg(pprev, cvec, w2row, src2d, dst2d, norm2d, parts,
            sp_h2, sp_agg, sidx, didx, nrm, gidx, didx2, vals,
            p0b, p1b, cb, wb, semL, semG, semS):
    cid = lax.axis_index("c")
    sid = lax.axis_index("s")
    wid = _wid()
    row0 = wid * (EPT // 128)

    def zb(i, _):
        p0b[pl.ds(i * L, L)] = jnp.zeros((L,), F32)
        return 0
    for k in range(K):
        off = k * N_PAD + sid * NPT
        pltpu.sync_copy(w2row.at[pl.ds(k * L, L)], wb)
        d1 = pltpu.async_copy(pprev.at[0, pl.ds(off, NPT)], p0b, semL)
        d2 = pltpu.async_copy(pprev.at[1, pl.ds(off, NPT)], p1b, semL)
        d3 = pltpu.async_copy(cvec.at[pl.ds(off, NPT)], cb, semL)
        d1.wait()
        d2.wait()
        d3.wait()
        wv = wb[pl.ds(0, L)]

        def mk(i, _):
            s = pl.ds(i * L, L)
            cb[s] = wv * (p0b[s] + p1b[s]) + cb[s]
            return 0
        lax.fori_loop(0, NPT // L, mk, 0)
        pltpu.sync_copy(cb, sp_h2.at[pl.ds(off, NPT)])
        lax.fori_loop(0, NPT // L, zb, 0)
        pltpu.sync_copy(p0b, sp_agg.at[pl.ds(off, NPT)])
    plsc.subcore_barrier()

    def fire_l(m, lp):
        base = jnp.minimum(row0 + m * MR4, jnp.int32(E2 - MR4))
        pltpu.async_copy(src2d.at[pl.ds(base, MR4)], sidx.at[lp], semL)
        pltpu.async_copy(dst2d.at[pl.ds(base, MR4)], didx.at[lp], semL)
        pltpu.async_copy(norm2d.at[pl.ds(base, MR4)], nrm.at[lp], semL)

    def wait_l():
        for _ in range(3):
            pltpu.make_async_copy(
                src2d.at[pl.ds(0, MR4)], sidx.at[0], semL).wait()

    def addk(lp, k):
        koff = jnp.int32(k * N_PAD)

        def body(r, _):
            for c in range(8):
                s = pl.ds(c * L, L)
                gidx[k, r, s] = sidx[lp, r, s] + koff
            return 0
        lax.fori_loop(0, MR4, body, 0)

    def fire_g(k):
        for j in range(MR4):
            pltpu.async_copy(sp_h2.at[gidx.at[k].at[j]], vals.at[k].at[j],
                             semG)

    def drain_g(k):
        for j in range(MR4):
            pltpu.make_async_copy(sp_h2.at[gidx.at[k].at[j]],
                                  vals.at[k].at[j], semG).wait()

    def mul(lp, k):
        koff = jnp.int32(k * N_PAD)

        def body(r, _):
            for c in range(8):
                s = pl.ds(c * L, L)
                vals[k, r, s] = vals[k, r, s] * nrm[lp, r, s]
                didx2[k, r, s] = didx[lp, r, s] + koff
            return 0
        lax.fori_loop(0, MR4, body, 0)

    def fire_s(k):
        for j in range(MR4):
            pltpu.async_copy(vals.at[k].at[j], sp_agg.at[didx2.at[k].at[j]],
                             semS, add=True)

    def drain_s(k):
        for j in range(MR4):
            pltpu.make_async_copy(vals.at[k].at[j],
                                  sp_agg.at[didx2.at[k].at[j]], semS).wait()

    fire_l(0, 0)
    wait_l()
    fire_l(1, 1)
    for k in range(K):
        addk(0, k)
        fire_g(k)
    for k in range(K):
        drain_g(k)
        mul(0, k)
        fire_s(k)

    def pair(t, _):
        for u in range(2):
            lp = (1 + u) % 2
            m = t * 2 + 1 + u
            wait_l()
            fire_l_base = jnp.minimum(row0 + (m + 1) * MR4,
                                      jnp.int32(E2 - MR4))
            pltpu.async_copy(src2d.at[pl.ds(fire_l_base, MR4)],
                             sidx.at[(m + 1) % 2], semL)
            pltpu.async_copy(dst2d.at[pl.ds(fire_l_base, MR4)],
                             didx.at[(m + 1) % 2], semL)
            pltpu.async_copy(norm2d.at[pl.ds(fire_l_base, MR4)],
                             nrm.at[(m + 1) % 2], semL)
            for k in range(K):
                addk(lp, k)
                drain_s(k)
                fire_g(k)
            for k in range(K):
                drain_g(k)
                mul(lp, k)
                fire_s(k)
        return 0
    lax.fori_loop(0, (NM4 - 1) // 2, pair, 0)

    wait_l()
    for k in range(K):
        drain_s(k)
    plsc.subcore_barrier()
    for k in range(K):
        off = k * N_PAD + sid * NPT
        pltpu.sync_copy(sp_agg.at[pl.ds(off, NPT)],
                        parts.at[cid, pl.ds(off, NPT)])


NP8 = N_PAD // 8
_T1B = 512
_NBLK1 = N_PAD // _T1B
_T2B = 1792
_NBLK2 = NP8 // _T2B


def _t1_body(xg_ref, rw_ref, iw_ref, b_ref, deg_ref, root_ref, h0_ref,
             dis_ref):
    xb = xg_ref[...]
    for k in range(K):
        root_ref[k] = jnp.dot(xb, rw_ref[k], preferred_element_type=F32) \
            + b_ref[k]
        h0_ref[k] = jnp.dot(xb, iw_ref[k], preferred_element_type=F32)
    d = deg_ref[0] + deg_ref[1]
    dis_ref[...] = jnp.where(d > 0.0, lax.rsqrt(jnp.abs(d) + 1e-30), 0.0)


_T1R = 1792


def _t1_call(xg, rwb, iwb, bb, deg2d):
    return pl.pallas_call(
        _t1_body,
        grid=(NP8 // _T1R,),
        in_specs=[
            pl.BlockSpec((_T1R, 8 * F_IN), lambda i: (i, 0)),
            pl.BlockSpec((K, 8 * F_IN, 128), lambda i: (0, 0, 0)),
            pl.BlockSpec((K, 8 * F_IN, 128), lambda i: (0, 0, 0)),
            pl.BlockSpec((K, 1, 128), lambda i: (0, 0, 0)),
            pl.BlockSpec((NC, _T1R // 16, 128), lambda i: (0, i, 0)),
        ],
        out_specs=[
            pl.BlockSpec((K, _T1R, 128), lambda i: (0, i, 0)),
            pl.BlockSpec((K, _T1R, 128), lambda i: (0, i, 0)),
            pl.BlockSpec((_T1R // 16, 128), lambda i: (i, 0)),
        ],
        out_shape=[
            jax.ShapeDtypeStruct((K, NP8, 128), F32),
            jax.ShapeDtypeStruct((K, NP8, 128), F32),
            jax.ShapeDtypeStruct((N_PAD // 128, 128), F32),
        ],
    )(xg, rwb, iwb, bb, deg2d)


def _t2_body(p_ref, root_ref, w_ref, h_ref):
    for k in range(K):
        out = jnp.maximum(p_ref[0, k] + p_ref[1, k] + root_ref[k], 0.0)
        h_ref[k] = jnp.dot(out, w_ref[k], preferred_element_type=F32)


def _t2_call(parts, rootb, w128):
    return pl.pallas_call(
        _t2_body,
        grid=(_NBLK2,),
        in_specs=[
            pl.BlockSpec((NC, K, _T2B, 128), lambda i: (0, 0, i, 0)),
            pl.BlockSpec((K, _T2B, 128), lambda i: (0, i, 0)),
            pl.BlockSpec((K, 128, 128), lambda i: (0, 0, 0)),
        ],
        out_specs=pl.BlockSpec((K, _T2B, 128), lambda i: (0, i, 0)),
        out_shape=jax.ShapeDtypeStruct((K, NP8, 128), F32),
    )(parts, rootb, w128)


def _t2b_body(p_ref, root_ref, sc_ref, sh_ref, rw2_ref, iw2_ref, b2_ref,
              w2_ref, root2_ref, h20_ref, c1_ref):
    acc = jnp.zeros((_T2B, 128), F32)
    for k in range(K):
        acc = acc + jnp.maximum(p_ref[0, k] + p_ref[1, k] + root_ref[k], 0.0)
    hm = acc * (1.0 / K)
    hbn = jnp.maximum(hm * sc_ref[...] + sh_ref[...], 0.0)
    for k in range(K):
        r2 = jnp.dot(hbn, rw2_ref[k], preferred_element_type=F32) \
            + b2_ref[k, 0, 0]
        h2 = jnp.dot(hbn, iw2_ref[k], preferred_element_type=F32)
        root2_ref[k] = r2
        h20_ref[k] = h2
        c1_ref[k] = r2 * w2_ref[k, 0, 0]


def _t2b_call(parts, rootb, bn_scale, bn_shift, rw2b, iw2b, b2, w2):
    return pl.pallas_call(
        _t2b_body,
        grid=(_NBLK2,),
        in_specs=[
            pl.BlockSpec((NC, K, _T2B, 128), lambda i: (0, 0, i, 0)),
            pl.BlockSpec((K, _T2B, 128), lambda i: (0, i, 0)),
            pl.BlockSpec((1, 128), lambda i: (0, 0)),
            pl.BlockSpec((1, 128), lambda i: (0, 0)),
            pl.BlockSpec((K, 128, 8), lambda i: (0, 0, 0)),
            pl.BlockSpec((K, 128, 8), lambda i: (0, 0, 0)),
            pl.BlockSpec((K, 1, 1), lambda i: (0, 0, 0)),
            pl.BlockSpec((K, 1, 1), lambda i: (0, 0, 0)),
        ],
        out_specs=[
            pl.BlockSpec((K, _T2B, 8), lambda i: (0, i, 0)),
            pl.BlockSpec((K, _T2B, 8), lambda i: (0, i, 0)),
            pl.BlockSpec((K, _T2B, 8), lambda i: (0, i, 0)),
        ],
        out_shape=[
            jax.ShapeDtypeStruct((K, NP8, 8), F32),
            jax.ShapeDtypeStruct((K, NP8, 8), F32),
            jax.ShapeDtypeStruct((K, NP8, 8), F32),
        ],
    )(parts, rootb, bn_scale, bn_shift, rw2b, iw2b, b2, w2)


NR128 = N_PAD // 128


def _f1_body(p_ref, root2_ref, o_ref):
    s = jnp.zeros((NR128, 128), F32)
    for k in range(K):
        s = s + p_ref[0, k] + p_ref[1, k] + root2_ref[k]
    o_ref[...] = jax.nn.sigmoid(s * (1.0 / K))


def _f1_call(parts, root2b):
    return pl.pallas_call(
        _f1_body,
        in_specs=[
            pl.BlockSpec((NC, K, NR128, 128), lambda: (0, 0, 0, 0)),
            pl.BlockSpec((K, NR128, 128), lambda: (0, 0, 0)),
        ],
        out_specs=pl.BlockSpec((NR128, 128), lambda: (0, 0)),
        out_shape=jax.ShapeDtypeStruct((NR128, 128), F32),
    )(parts, root2b)


def kernel(x, edge_index, edge_attr, batch,
           conv1_init_w, conv1_w, conv1_root_w, conv1_bias,
           bn1_gamma, bn1_beta, bn1_mean, bn1_var,
           conv2_init_w, conv2_w, conv2_root_w, conv2_bias):
    del batch
    pad = E_PAD - E
    fill = (jnp.arange(pad, dtype=I32) * 37) % N
    src = jnp.concatenate([edge_index[0].astype(I32), fill]).reshape(E2, 128)
    dst = jnp.concatenate([edge_index[1].astype(I32), fill]).reshape(E2, 128)
    ea = jnp.concatenate([edge_attr.astype(F32),
                          jnp.zeros((pad,), F32)]).reshape(E2, 128)

    eye8 = jnp.eye(8, dtype=F32)
    xg = x.reshape(NP8 // 196 * 196, 8 * F_IN) if False else \
        jnp.pad(x, ((0, N_PAD - N), (0, 0))).reshape(NP8, 8 * F_IN)
    rwb1 = jnp.einsum("ab,kij->kaibj", eye8,
                      conv1_root_w).reshape(K, 8 * F_IN, 128)
    iwb1 = jnp.einsum("ab,kij->kaibj", eye8,
                      conv1_init_w).reshape(K, 8 * F_IN, 128)
    bb1 = jnp.tile(conv1_bias, (1, 1, 8)).reshape(K, 1, 128)

    deg_parts = _s1_deg(dst, ea)
    rootb, h0, dis = _t1_call(xg, rwb1, iwb1, bb1,
                              deg_parts.reshape(NC, N_PAD // 128, 128))
    norm2d = _s2_norm(dis.reshape(N_PAD), src, dst, ea)

    w128 = jnp.einsum("ab,kij->kaibj", eye8, conv1_w).reshape(K, 128, 128)
    rw2b = jnp.einsum("ab,kij->kaibj", eye8, conv2_root_w).reshape(K, 128, 8)
    iw2b = jnp.einsum("ab,kij->kaibj", eye8, conv2_init_w).reshape(K, 128, 8)
    bn_scale16 = bn1_gamma * lax.rsqrt(bn1_var + 1e-5)
    bn_shift16 = bn1_beta - bn1_mean * bn_scale16
    bn_scale = jnp.tile(bn_scale16, 8).reshape(1, 128)
    bn_shift = jnp.tile(bn_shift16, 8).reshape(1, 128)

    rootb_p = rootb
    h = h0.reshape(K * N_PAD, HID)
    for _ in range(T - 1):
        parts1 = _s3_msg(h, src, dst, norm2d)
        h = _t2_call(parts1.reshape(NC, K, NP8, 128), rootb_p,
                     w128).reshape(K * N_PAD, HID)
    parts1 = _s3_msg(h, src, dst, norm2d)
    root2b, h20, c1 = _t2b_call(
        parts1.reshape(NC, K, NP8, 128), rootb_p, bn_scale, bn_shift,
        rw2b, iw2b, conv2_bias, conv2_w)

    w2row = jnp.broadcast_to(
        conv2_w.reshape(K, 1).astype(F32), (K, L)).reshape(K * L)
    zeros2 = jnp.zeros((NC, K * N_PAD), F32)
    cvec = h20.reshape(K * N_PAD)
    c1f = c1.reshape(K * N_PAD)
    parts2 = _s4_msg(zeros2, cvec, w2row, src, dst, norm2d)
    for _ in range(T - 1):
        parts2 = _s4_msg(parts2, c1f, w2row, src, dst, norm2d)

    out = _f1_call(parts2.reshape(NC, K, NR128, 128),
                   root2b.reshape(K, NR128, 128))
    return out.reshape(N_PAD)[:N].reshape(N, 1)
